# baseline (speedup 1.0000x reference)
import functools
import jax
import jax.numpy as jnp
from jax import lax
from jax.experimental import pallas as pl
from jax.experimental.pallas import tpu as pltpu
from jax.experimental.pallas import tpu_sc as plsc

N = 10000
E = 320000
D = 128
H = 64
NT = 16
EPT = E // NT
CH = 40
NCHUNK = EPT // CH
NPAD = 10240
RPT = NPAD // NT
F32 = jnp.float32


def _half_cols(c, hbm_ref, off, n, vbuf, to_hbm=False, sem=None):
    for cc in (0, 1):
        @pl.when(c == cc)
        def _():
            sl = hbm_ref.at[pl.ds(off, n), pl.ds(cc * H, H)]
            if sem is None:
                if to_hbm:
                    pltpu.sync_copy(vbuf, sl)
                else:
                    pltpu.sync_copy(sl, vbuf)
            elif to_hbm:
                pltpu.async_copy(vbuf, sl, sem)
            else:
                pltpu.async_copy(sl, vbuf, sem)


def _sc_body(want_e, src_h, dst_h, ebx_h, dx_h, ce_h, *rest):
    if want_e:
        eij_h, nd_h, stats_h = rest[:3]
        scr = rest[3:]
    else:
        nd_h = rest[0]
        scr = rest[1:]
    (srcv0, dstv0, ebxv0, dxv0, cev0, eijv0, ms0,
     srcv1, dstv1, ebxv1, dxv1, cev1, eijv1, ms1,
     statsv, nd_s, gsem0, gsem1, osem0, osem1) = scr

    c = lax.axis_index("c")
    s = lax.axis_index("s")

    def zrow(r, carry):
        for j in range(8):
            ms0[r, pl.ds(j * 16, 16)] = jnp.zeros((16,), F32)
        return carry

    lax.fori_loop(0, CH, zrow, 0)
    for k in range(RPT // CH):
        pltpu.sync_copy(ms0, nd_s.at[pl.ds(s * RPT + k * CH, CH)])
    plsc.subcore_barrier()

    base = s * EPT
    ebxc = ebx_h.at[c]
    dxc = dx_h.at[c]
    sets = (
        (srcv0, dstv0, ebxv0, dxv0, cev0, eijv0, ms0, gsem0, osem0),
        (srcv1, dstv1, ebxv1, dxv1, cev1, eijv1, ms1, gsem1, osem1),
    )

    def make_row(st):
        srcv, dstv, ebxv, dxv, cev, eijv, ms, gsem, osem = st

        def row(r, rc):
            out = rc
            if want_e:
                sums = list(rc[:4])
                sqs = list(rc[4:])
            for j in range(4):
                sl = pl.ds(j * 16, 16)
                slh = pl.ds(H + j * 16, 16)
                eij = dxv[r, sl] + ebxv[r, sl] + cev[r, sl]
                sig = 1.0 / (1.0 + jnp.exp(-eij))
                ms[r, sl] = sig * ebxv[r, slh]
                ms[r, slh] = sig
                if want_e:
                    eijv[r, sl] = eij
                    sums[j] = sums[j] + eij
                    sqs[j] = sqs[j] + eij * eij
            if want_e:
                out = tuple(sums) + tuple(sqs)
            return out

        return row

    rows = tuple(make_row(st) for st in sets)

    def chunk(i, carry):
        srcv, dstv, ebxv, dxv, cev, eijv, ms, gsem, osem = sets[0]
        off = base + i * CH
        pltpu.sync_copy(src_h.at[pl.ds(off, CH)], srcv)
        pltpu.sync_copy(dst_h.at[pl.ds(off, CH)], dstv)
        d1 = pltpu.async_copy(ebxc.at[srcv], ebxv, gsem)
        d2 = pltpu.async_copy(dxc.at[dstv], dxv, gsem)
        _half_cols(c, ce_h, off, CH, cev)
        d1.wait()
        d2.wait()
        carry = lax.fori_loop(0, CH, rows[0], carry)
        if want_e:
            _half_cols(c, eij_h, off, CH, eijv, to_hbm=True)
        pltpu.sync_copy(ms, nd_s.at[dstv], add=True)
        return carry

    if want_e:
        init = tuple(jnp.zeros((16,), F32) for _ in range(8))
    else:
        init = 0
    fin = lax.fori_loop(0, NCHUNK, chunk, init)

    if want_e:
        for j in range(4):
            statsv[0, pl.ds(j * 16, 16)] = fin[j]
            statsv[1, pl.ds(j * 16, 16)] = fin[4 + j]
        pltpu.sync_copy(statsv, stats_h.at[c, s])

    plsc.subcore_barrier()
    for k in range(RPT // CH):
        r0 = s * RPT + k * CH
        pltpu.sync_copy(nd_s.at[pl.ds(r0, CH)], ms0)
        pltpu.sync_copy(ms0, nd_h.at[c, pl.ds(r0, CH)])


def _make_sc_kernel(want_e):
    outs = []
    if want_e:
        outs.append(jax.ShapeDtypeStruct((E, D), F32))
    outs.append(jax.ShapeDtypeStruct((2, NPAD, D), F32))
    if want_e:
        outs.append(jax.ShapeDtypeStruct((2, NT, 2, H), F32))
    bufset = [
        pltpu.VMEM((CH,), jnp.int32),
        pltpu.VMEM((CH,), jnp.int32),
        pltpu.VMEM((CH, D), F32),
        pltpu.VMEM((CH, H), F32),
        pltpu.VMEM((CH, H), F32),
        pltpu.VMEM((CH, H), F32),
        pltpu.VMEM((CH, D), F32),
    ]
    scratch = bufset + bufset + [
        pltpu.VMEM((2, H), F32),
        pltpu.VMEM_SHARED((NPAD, D), F32),
        pltpu.SemaphoreType.DMA,
        pltpu.SemaphoreType.DMA,
        pltpu.SemaphoreType.DMA,
        pltpu.SemaphoreType.DMA,
    ]
    mesh = plsc.VectorSubcoreMesh(core_axis_name="c", subcore_axis_name="s")
    return pl.kernel(
        functools.partial(_sc_body, want_e),
        out_type=tuple(outs),
        mesh=mesh,
        scratch_types=scratch,
        compiler_params=pltpu.CompilerParams(use_tc_tiling_on_sc=False),
    )


NBLK = 1000
EBLK = 4000


def _gelu(x):
    return 0.5 * x * (1.0 + lax.erf(x * 0.7071067811865476))


def _proj_body(x_ref, w_ref, b_ref, ax_ref, dx_ref, ebx_ref):
    p = jnp.dot(x_ref[...], w_ref[...], preferred_element_type=F32) + b_ref[...]
    ax_ref[...] = p[:, :D]
    bx = p[:, D:2 * D]
    dx = p[:, 2 * D:3 * D]
    ex = p[:, 3 * D:4 * D]
    dx_ref[...] = jnp.stack([dx[:, :H], dx[:, H:]], axis=0)
    ebx_ref[...] = jnp.stack(
        [jnp.concatenate([ex[:, :H], bx[:, :H]], axis=1),
         jnp.concatenate([ex[:, H:], bx[:, H:]], axis=1)], axis=0)


def _node_proj(x, wcat, bcat):
    grid = N // NBLK
    return pl.pallas_call(
        _proj_body,
        grid=(grid,),
        in_specs=[
            pl.BlockSpec((NBLK, D), lambda i: (i, 0)),
            pl.BlockSpec((D, 4 * D), lambda i: (0, 0)),
            pl.BlockSpec((1, 4 * D), lambda i: (0, 0)),
        ],
        out_specs=[
            pl.BlockSpec((NBLK, D), lambda i: (i, 0)),
            pl.BlockSpec((2, NBLK, H), lambda i: (0, i, 0)),
            pl.BlockSpec((2, NBLK, D), lambda i: (0, i, 0)),
        ],
        out_shape=[
            jax.ShapeDtypeStruct((N, D), F32),
            jax.ShapeDtypeStruct((2, N, H), F32),
            jax.ShapeDtypeStruct((2, N, D), F32),
        ],
    )(x, wcat, bcat)


def _bond_ce_body(attr_ref, tbl_ref, wc_ref, bc_ref, e0_ref, ce_ref):
    attr = attr_ref[...]
    ohs = []
    for k in range(3):
        iota = lax.broadcasted_iota(jnp.int32, (1, 5), 1)
        ohs.append((attr[:, k:k + 1] == iota).astype(F32))
    oh = jnp.concatenate(ohs, axis=1)
    e0 = jnp.dot(oh, tbl_ref[...], preferred_element_type=F32)
    e0_ref[...] = e0
    ce_ref[...] = jnp.dot(e0, wc_ref[...], preferred_element_type=F32) + bc_ref[...]


def _bond_ce(edge_attr, tblcat, wc, bc):
    grid = E // EBLK
    return pl.pallas_call(
        _bond_ce_body,
        grid=(grid,),
        in_specs=[
            pl.BlockSpec((EBLK, 3), lambda i: (i, 0)),
            pl.BlockSpec((15, D), lambda i: (0, 0)),
            pl.BlockSpec((D, D), lambda i: (0, 0)),
            pl.BlockSpec((1, D), lambda i: (0, 0)),
        ],
        out_specs=[
            pl.BlockSpec((EBLK, D), lambda i: (i, 0)),
            pl.BlockSpec((EBLK, D), lambda i: (i, 0)),
        ],
        out_shape=[
            jax.ShapeDtypeStruct((E, D), F32),
            jax.ShapeDtypeStruct((E, D), F32),
        ],
    )(edge_attr, tblcat, wc, bc)


def _edge_update_ce_body(eij_ref, ep_ref, st_ref, g_ref, b_ref, wc_ref,
                         bc_ref, e_ref, ce_ref):
    st = st_ref[...]
    sums = jnp.sum(st[:, :, 0, :], axis=1)
    sqs = jnp.sum(st[:, :, 1, :], axis=1)
    mu = jnp.concatenate([sums[0], sums[1]])[None, :] * (1.0 / E)
    var = jnp.concatenate([sqs[0], sqs[1]])[None, :] * (1.0 / E) - mu * mu
    inv = lax.rsqrt(var + 1e-5)
    xn = (eij_ref[...] - mu) * inv * g_ref[...] + b_ref[...]
    xn = _gelu(xn)
    enew = ep_ref[...] + xn
    e_ref[...] = enew
    ce_ref[...] = jnp.dot(enew, wc_ref[...], preferred_element_type=F32) + bc_ref[...]


def _edge_update_ce(eij, e_prev, stats, gamma, beta, wc, bc):
    grid = E // EBLK
    return pl.pallas_call(
        _edge_update_ce_body,
        grid=(grid,),
        in_specs=[
            pl.BlockSpec((EBLK, D), lambda i: (i, 0)),
            pl.BlockSpec((EBLK, D), lambda i: (i, 0)),
            pl.BlockSpec((2, NT, 2, H), lambda i: (0, 0, 0, 0)),
            pl.BlockSpec((1, D), lambda i: (0, 0)),
            pl.BlockSpec((1, D), lambda i: (0, 0)),
            pl.BlockSpec((D, D), lambda i: (0, 0)),
            pl.BlockSpec((1, D), lambda i: (0, 0)),
        ],
        out_specs=[
            pl.BlockSpec((EBLK, D), lambda i: (i, 0)),
            pl.BlockSpec((EBLK, D), lambda i: (i, 0)),
        ],
        out_shape=[
            jax.ShapeDtypeStruct((E, D), F32),
            jax.ShapeDtypeStruct((E, D), F32),
        ],
    )(eij, e_prev, stats, gamma, beta, wc, bc)


def _node_update_body(ax_ref, nd_ref, xin_ref, g_ref, b_ref, out_ref):
    num0 = nd_ref[0, :N, :H]
    num1 = nd_ref[1, :N, :H]
    den0 = nd_ref[0, :N, H:]
    den1 = nd_ref[1, :N, H:]
    aggr = jnp.concatenate(
        [num0 / (den0 + 1e-6), num1 / (den1 + 1e-6)], axis=1)
    xn = ax_ref[...] + aggr
    mu = jnp.mean(xn, axis=0, keepdims=True)
    var = jnp.mean((xn - mu) ** 2, axis=0, keepdims=True)
    xn = (xn - mu) * lax.rsqrt(var + 1e-5) * g_ref[...] + b_ref[...]
    out_ref[...] = xin_ref[...] + _gelu(xn)


def _node_update(ax, nd, x_in, gamma, beta):
    return pl.pallas_call(
        _node_update_body,
        out_shape=jax.ShapeDtypeStruct((N, D), F32),
    )(ax, nd, x_in, gamma, beta)


def kernel(X_n, edge_index, edge_attr, PE, params):
    src = edge_index[0]
    dst = edge_index[1]
    tblcat = params["bond_tables"].reshape(3 * 5, D)
    layers = params["layers"]

    es, ce = _bond_ce(edge_attr, tblcat, layers[0]["WC"],
                      layers[0]["bC"][None, :])
    x = X_n
    for l, lp in enumerate(layers):
        wcat = jnp.concatenate([lp["WA"], lp["WB"], lp["WD"], lp["WE"]],
                               axis=1)
        bcat = jnp.concatenate([lp["bA"], lp["bB"], lp["bD"], lp["bE"]])[None, :]
        ax, dx3, ebx = _node_proj(x, wcat, bcat)
        want_e = l + 1 < len(layers)
        sc = _make_sc_kernel(want_e)
        if want_e:
            eijs, nd, stats = sc(src, dst, ebx, dx3, ce)
        else:
            (nd,) = sc(src, dst, ebx, dx3, ce)
        x = _node_update(ax, nd, x, lp["gamma_x"][None, :],
                         lp["beta_x"][None, :])
        if want_e:
            nlp = layers[l + 1]
            es, ce = _edge_update_ce(eijs, es, stats, lp["gamma_e"][None, :],
                                     lp["beta_e"][None, :], nlp["WC"],
                                     nlp["bC"][None, :])
    return x


# speedup vs baseline: 1.1405x; 1.1405x over previous
import functools
import jax
import jax.numpy as jnp
from jax import lax
from jax.experimental import pallas as pl
from jax.experimental.pallas import tpu as pltpu
from jax.experimental.pallas import tpu_sc as plsc

N = 10000
E = 320000
D = 128
H = 64
NT = 16
EPT = E // NT
CH = 40
NCHUNK = EPT // CH
NPAD = 10240
RPT = NPAD // NT
F32 = jnp.float32


def _half_cols(c, hbm_ref, off, n, vbuf, to_hbm=False, sem=None):
    for cc in (0, 1):
        @pl.when(c == cc)
        def _():
            sl = hbm_ref.at[pl.ds(off, n), pl.ds(cc * H, H)]
            if sem is None:
                if to_hbm:
                    pltpu.sync_copy(vbuf, sl)
                else:
                    pltpu.sync_copy(sl, vbuf)
            elif to_hbm:
                pltpu.async_copy(vbuf, sl, sem)
            else:
                pltpu.async_copy(sl, vbuf, sem)


def _sc_body(want_e, src_h, dst2_h, ebx_h, dx_h, ce_h, *rest):
    if want_e:
        eij_h, nd_h, stats_h = rest[:3]
        scr = rest[3:]
    else:
        nd_h = rest[0]
        scr = rest[1:]
    (ebxv0, dxv0, cev0, ms0, ebxv1, dxv1, cev1, ms1, sidx, didx,
     statsv, nd_s, gsem0, gsem1, cesem0, cesem1, osem0, osem1) = scr

    c = lax.axis_index("c")
    s = lax.axis_index("s")

    def zrow(r, carry):
        for j in range(8):
            ms0[r, pl.ds(j * 16, 16)] = jnp.zeros((16,), F32)
        return carry

    lax.fori_loop(0, CH, zrow, 0)
    for k in range(RPT // CH):
        pltpu.sync_copy(ms0, nd_s.at[pl.ds(s * RPT + k * CH, CH)])
    plsc.subcore_barrier()

    base = s * EPT
    ebxc = ebx_h.at[c]
    dxc = dx_h.at[c]
    sets = (
        (ebxv0, dxv0, cev0, ms0, gsem0, cesem0, osem0),
        (ebxv1, dxv1, cev1, ms1, gsem1, cesem1, osem1),
    )

    def make_row(st):
        ebxv, dxv, cev, ms = st[:4]

        def row(r, rc):
            out = rc
            if want_e:
                sums = list(rc[:4])
                sqs = list(rc[4:])
            for j in range(4):
                sl = pl.ds(j * 16, 16)
                slh = pl.ds(H + j * 16, 16)
                eij = dxv[r, sl] + ebxv[r, sl] + cev[r, sl]
                sig = 1.0 / (1.0 + jnp.exp(-eij))
                ms[r, sl] = sig * ebxv[r, slh]
                ms[r, slh] = sig
                cev[r, sl] = eij
                if want_e:
                    sums[j] = sums[j] + eij
                    sqs[j] = sqs[j] + eij * eij
            if want_e:
                out = tuple(sums) + tuple(sqs)
            return out

        return row

    rows = tuple(make_row(st) for st in sets)

    def body(k, carry):
        off0 = base + k * 2 * CH
        pltpu.sync_copy(src_h.at[pl.ds(off0, 2 * CH)], sidx)
        pltpu.sync_copy(dst2_h.at[pl.ds(s * (NCHUNK // 2) + k, 1)], didx)
        gd = []
        for p in (0, 1):
            ebxv, dxv, cev, ms, gsem, cesem, osem = sets[p]
            gd.append(pltpu.async_copy(ebxc.at[sidx.at[pl.ds(p * CH, CH)]],
                                       ebxv, gsem))
            gd.append(pltpu.async_copy(dxc.at[didx.at[0, p]], dxv, gsem))
            _half_cols(c, ce_h, off0 + p * CH, CH, cev, sem=cesem)
        sd = []
        for p in (0, 1):
            ebxv, dxv, cev, ms, gsem, cesem, osem = sets[p]
            gd[2 * p].wait()
            gd[2 * p + 1].wait()
            pltpu.make_async_copy(
                ce_h.at[pl.ds(0, CH), pl.ds(0, H)], cev, cesem).wait()
            carry = lax.fori_loop(0, CH, rows[p], carry)
            if want_e:
                _half_cols(c, eij_h, off0 + p * CH, CH, cev, to_hbm=True,
                           sem=osem)
            pltpu.sync_copy(ms, nd_s.at[didx.at[0, p]], add=True)
        for p in (0, 1):
            ebxv, dxv, cev, ms, gsem, cesem, osem = sets[p]
            if want_e:
                pltpu.make_async_copy(
                    cev, eij_h.at[pl.ds(0, CH), pl.ds(0, H)], osem).wait()
        return carry

    if want_e:
        init = tuple(jnp.zeros((16,), F32) for _ in range(8))
    else:
        init = 0
    fin = lax.fori_loop(0, NCHUNK // 2, body, init)

    if want_e:
        for j in range(4):
            statsv[0, pl.ds(j * 16, 16)] = fin[j]
            statsv[1, pl.ds(j * 16, 16)] = fin[4 + j]
        pltpu.sync_copy(statsv, stats_h.at[c, s])

    plsc.subcore_barrier()
    for k in range(RPT // CH):
        r0 = s * RPT + k * CH
        pltpu.sync_copy(nd_s.at[pl.ds(r0, CH)], ms0)
        pltpu.sync_copy(ms0, nd_h.at[c, pl.ds(r0, CH)])


def _make_sc_kernel(want_e):
    outs = []
    if want_e:
        outs.append(jax.ShapeDtypeStruct((E, D), F32))
    outs.append(jax.ShapeDtypeStruct((2, NPAD, D), F32))
    if want_e:
        outs.append(jax.ShapeDtypeStruct((2, NT, 2, H), F32))
    bufset = [
        pltpu.VMEM((CH, D), F32),
        pltpu.VMEM((CH, H), F32),
        pltpu.VMEM((CH, H), F32),
        pltpu.VMEM((CH, D), F32),
    ]
    scratch = bufset + bufset + [
        pltpu.VMEM((2 * CH,), jnp.int32),
        pltpu.VMEM((1, 2, CH), jnp.int32),
        pltpu.VMEM((2, H), F32),
        pltpu.VMEM_SHARED((NPAD, D), F32),
        pltpu.SemaphoreType.DMA,
        pltpu.SemaphoreType.DMA,
        pltpu.SemaphoreType.DMA,
        pltpu.SemaphoreType.DMA,
        pltpu.SemaphoreType.DMA,
        pltpu.SemaphoreType.DMA,
    ]
    mesh = plsc.VectorSubcoreMesh(core_axis_name="c", subcore_axis_name="s")
    return pl.kernel(
        functools.partial(_sc_body, want_e),
        out_type=tuple(outs),
        mesh=mesh,
        scratch_types=scratch,
        compiler_params=pltpu.CompilerParams(use_tc_tiling_on_sc=False),
    )


NBLK = 1000
EBLK = 4000


def _gelu(x):
    return 0.5 * x * (1.0 + lax.erf(x * 0.7071067811865476))


def _proj_body(x_ref, w_ref, b_ref, ax_ref, dx_ref, ebx_ref):
    p = jnp.dot(x_ref[...], w_ref[...], preferred_element_type=F32) + b_ref[...]
    ax_ref[...] = p[:, :D]
    bx = p[:, D:2 * D]
    dx = p[:, 2 * D:3 * D]
    ex = p[:, 3 * D:4 * D]
    dx_ref[...] = jnp.stack([dx[:, :H], dx[:, H:]], axis=0)
    ebx_ref[...] = jnp.stack(
        [jnp.concatenate([ex[:, :H], bx[:, :H]], axis=1),
         jnp.concatenate([ex[:, H:], bx[:, H:]], axis=1)], axis=0)


def _node_proj(x, wcat, bcat):
    grid = N // NBLK
    return pl.pallas_call(
        _proj_body,
        grid=(grid,),
        in_specs=[
            pl.BlockSpec((NBLK, D), lambda i: (i, 0)),
            pl.BlockSpec((D, 4 * D), lambda i: (0, 0)),
            pl.BlockSpec((1, 4 * D), lambda i: (0, 0)),
        ],
        out_specs=[
            pl.BlockSpec((NBLK, D), lambda i: (i, 0)),
            pl.BlockSpec((2, NBLK, H), lambda i: (0, i, 0)),
            pl.BlockSpec((2, NBLK, D), lambda i: (0, i, 0)),
        ],
        out_shape=[
            jax.ShapeDtypeStruct((N, D), F32),
            jax.ShapeDtypeStruct((2, N, H), F32),
            jax.ShapeDtypeStruct((2, N, D), F32),
        ],
    )(x, wcat, bcat)


def _bond_ce_body(attr_ref, tbl_ref, wc_ref, bc_ref, e0_ref, ce_ref):
    attr = attr_ref[...]
    ohs = []
    for k in range(3):
        iota = lax.broadcasted_iota(jnp.int32, (1, 5), 1)
        ohs.append((attr[:, k:k + 1] == iota).astype(F32))
    oh = jnp.concatenate(ohs, axis=1)
    e0 = jnp.dot(oh, tbl_ref[...], preferred_element_type=F32)
    e0_ref[...] = e0
    ce_ref[...] = jnp.dot(e0, wc_ref[...], preferred_element_type=F32) + bc_ref[...]


def _bond_ce(edge_attr, tblcat, wc, bc):
    grid = E // EBLK
    return pl.pallas_call(
        _bond_ce_body,
        grid=(grid,),
        in_specs=[
            pl.BlockSpec((EBLK, 3), lambda i: (i, 0)),
            pl.BlockSpec((15, D), lambda i: (0, 0)),
            pl.BlockSpec((D, D), lambda i: (0, 0)),
            pl.BlockSpec((1, D), lambda i: (0, 0)),
        ],
        out_specs=[
            pl.BlockSpec((EBLK, D), lambda i: (i, 0)),
            pl.BlockSpec((EBLK, D), lambda i: (i, 0)),
        ],
        out_shape=[
            jax.ShapeDtypeStruct((E, D), F32),
            jax.ShapeDtypeStruct((E, D), F32),
        ],
    )(edge_attr, tblcat, wc, bc)


def _edge_update_ce_body(eij_ref, ep_ref, st_ref, g_ref, b_ref, wc_ref,
                         bc_ref, e_ref, ce_ref):
    st = st_ref[...]
    sums = jnp.sum(st[:, :, 0, :], axis=1)
    sqs = jnp.sum(st[:, :, 1, :], axis=1)
    mu = jnp.concatenate([sums[0], sums[1]])[None, :] * (1.0 / E)
    var = jnp.concatenate([sqs[0], sqs[1]])[None, :] * (1.0 / E) - mu * mu
    inv = lax.rsqrt(var + 1e-5)
    xn = (eij_ref[...] - mu) * inv * g_ref[...] + b_ref[...]
    xn = _gelu(xn)
    enew = ep_ref[...] + xn
    e_ref[...] = enew
    ce_ref[...] = jnp.dot(enew, wc_ref[...], preferred_element_type=F32) + bc_ref[...]


def _edge_update_ce(eij, e_prev, stats, gamma, beta, wc, bc):
    grid = E // EBLK
    return pl.pallas_call(
        _edge_update_ce_body,
        grid=(grid,),
        in_specs=[
            pl.BlockSpec((EBLK, D), lambda i: (i, 0)),
            pl.BlockSpec((EBLK, D), lambda i: (i, 0)),
            pl.BlockSpec((2, NT, 2, H), lambda i: (0, 0, 0, 0)),
            pl.BlockSpec((1, D), lambda i: (0, 0)),
            pl.BlockSpec((1, D), lambda i: (0, 0)),
            pl.BlockSpec((D, D), lambda i: (0, 0)),
            pl.BlockSpec((1, D), lambda i: (0, 0)),
        ],
        out_specs=[
            pl.BlockSpec((EBLK, D), lambda i: (i, 0)),
            pl.BlockSpec((EBLK, D), lambda i: (i, 0)),
        ],
        out_shape=[
            jax.ShapeDtypeStruct((E, D), F32),
            jax.ShapeDtypeStruct((E, D), F32),
        ],
    )(eij, e_prev, stats, gamma, beta, wc, bc)


def _node_update_body(ax_ref, nd_ref, xin_ref, g_ref, b_ref, out_ref):
    num0 = nd_ref[0, :N, :H]
    num1 = nd_ref[1, :N, :H]
    den0 = nd_ref[0, :N, H:]
    den1 = nd_ref[1, :N, H:]
    aggr = jnp.concatenate(
        [num0 / (den0 + 1e-6), num1 / (den1 + 1e-6)], axis=1)
    xn = ax_ref[...] + aggr
    mu = jnp.mean(xn, axis=0, keepdims=True)
    var = jnp.mean((xn - mu) ** 2, axis=0, keepdims=True)
    xn = (xn - mu) * lax.rsqrt(var + 1e-5) * g_ref[...] + b_ref[...]
    out_ref[...] = xin_ref[...] + _gelu(xn)


def _node_update(ax, nd, x_in, gamma, beta):
    return pl.pallas_call(
        _node_update_body,
        out_shape=jax.ShapeDtypeStruct((N, D), F32),
    )(ax, nd, x_in, gamma, beta)


def kernel(X_n, edge_index, edge_attr, PE, params):
    src = edge_index[0]
    dst = edge_index[1]
    tblcat = params["bond_tables"].reshape(3 * 5, D)
    layers = params["layers"]

    es, ce = _bond_ce(edge_attr, tblcat, layers[0]["WC"],
                      layers[0]["bC"][None, :])
    x = X_n
    for l, lp in enumerate(layers):
        wcat = jnp.concatenate([lp["WA"], lp["WB"], lp["WD"], lp["WE"]],
                               axis=1)
        bcat = jnp.concatenate([lp["bA"], lp["bB"], lp["bD"], lp["bE"]])[None, :]
        ax, dx3, ebx = _node_proj(x, wcat, bcat)
        want_e = l + 1 < len(layers)
        sc = _make_sc_kernel(want_e)
        dst2 = dst.reshape(NT * (NCHUNK // 2), 2, CH)
        if want_e:
            eijs, nd, stats = sc(src, dst2, ebx, dx3, ce)
        else:
            (nd,) = sc(src, dst2, ebx, dx3, ce)
        x = _node_update(ax, nd, x, lp["gamma_x"][None, :],
                         lp["beta_x"][None, :])
        if want_e:
            nlp = layers[l + 1]
            es, ce = _edge_update_ce(eijs, es, stats, lp["gamma_e"][None, :],
                                     lp["beta_e"][None, :], nlp["WC"],
                                     nlp["bC"][None, :])
    return x


# speedup vs baseline: 2.5001x; 2.1921x over previous
import functools
import jax
import jax.numpy as jnp
from jax import lax
from jax.experimental import pallas as pl
from jax.experimental.pallas import tpu as pltpu
from jax.experimental.pallas import tpu_sc as plsc

N = 10000
E = 320000
D = 128
H = 64
NT = 16
EPT = E // NT
CH = 40
NCHUNK = EPT // CH
NPAD = 10240
RPT = NPAD // NT
F32 = jnp.float32


def _half_cols(c, hbm_ref, off, n, vbuf, to_hbm=False, sem=None):
    for cc in (0, 1):
        @pl.when(c == cc)
        def _():
            sl = hbm_ref.at[pl.ds(off, n), pl.ds(cc * H, H)]
            if sem is None:
                if to_hbm:
                    pltpu.sync_copy(vbuf, sl)
                else:
                    pltpu.sync_copy(sl, vbuf)
            elif to_hbm:
                pltpu.async_copy(vbuf, sl, sem)
            else:
                pltpu.async_copy(sl, vbuf, sem)


def _sc_body(want_e, src_h, dst2_h, bx_h, dx_h, ex_h, ce_h, *rest):
    if want_e:
        eij_h, num_h, den_h, stats_h = rest[:4]
        scr = rest[4:]
    else:
        num_h, den_h = rest[:2]
        scr = rest[2:]
    (bxv0, dxv0, exv0, cev0, msgv0, sigv0,
     bxv1, dxv1, exv1, cev1, msgv1, sigv1,
     sidx, didx, statsv, num_s, den_s,
     gsem0, gsem1, cesem0, cesem1, osem0, osem1) = scr

    c = lax.axis_index("c")
    s = lax.axis_index("s")

    def zrow(r, carry):
        for j in range(4):
            msgv0[r, pl.ds(j * 16, 16)] = jnp.zeros((16,), F32)
        return carry

    lax.fori_loop(0, CH, zrow, 0)
    for k in range(RPT // CH):
        r0 = s * RPT + k * CH
        pltpu.sync_copy(msgv0, num_s.at[pl.ds(r0, CH)])
        pltpu.sync_copy(msgv0, den_s.at[pl.ds(r0, CH)])
    plsc.subcore_barrier()

    base = s * EPT
    bxc = bx_h.at[c]
    dxc = dx_h.at[c]
    exc = ex_h.at[c]
    sets = (
        (bxv0, dxv0, exv0, cev0, msgv0, sigv0, gsem0, cesem0, osem0),
        (bxv1, dxv1, exv1, cev1, msgv1, sigv1, gsem1, cesem1, osem1),
    )

    def make_row(st):
        bxv, dxv, exv, cev, msgv, sigv = st[:6]

        def row(r, rc):
            out = rc
            if want_e:
                sums = list(rc[:4])
                sqs = list(rc[4:])
            for j in range(4):
                sl = pl.ds(j * 16, 16)
                eij = dxv[r, sl] + exv[r, sl] + cev[r, sl]
                sig = 1.0 / (1.0 + jnp.exp(-eij))
                msgv[r, sl] = sig * bxv[r, sl]
                sigv[r, sl] = sig
                cev[r, sl] = eij
                if want_e:
                    sums[j] = sums[j] + eij
                    sqs[j] = sqs[j] + eij * eij
            if want_e:
                out = tuple(sums) + tuple(sqs)
            return out

        return row

    rows = tuple(make_row(st) for st in sets)

    def body(k, carry):
        off0 = base + k * 2 * CH
        pltpu.sync_copy(src_h.at[pl.ds(off0, 2 * CH)], sidx)
        pltpu.sync_copy(dst2_h.at[pl.ds(s * (NCHUNK // 2) + k, 1)], didx)
        gd = []
        for p in (0, 1):
            bxv, dxv, exv, cev, msgv, sigv, gsem, cesem, osem = sets[p]
            idxp = sidx.at[pl.ds(p * CH, CH)]
            gd.append(pltpu.async_copy(bxc.at[idxp], bxv, gsem))
            gd.append(pltpu.async_copy(exc.at[idxp], exv, gsem))
            gd.append(pltpu.async_copy(dxc.at[didx.at[0, p]], dxv, gsem))
            _half_cols(c, ce_h, off0 + p * CH, CH, cev, sem=cesem)
        for p in (0, 1):
            bxv, dxv, exv, cev, msgv, sigv, gsem, cesem, osem = sets[p]
            gd[3 * p].wait()
            gd[3 * p + 1].wait()
            gd[3 * p + 2].wait()
            pltpu.make_async_copy(
                ce_h.at[pl.ds(0, CH), pl.ds(0, H)], cev, cesem).wait()
            carry = lax.fori_loop(0, CH, rows[p], carry)
            if want_e:
                _half_cols(c, eij_h, off0 + p * CH, CH, cev, to_hbm=True,
                           sem=osem)
            pltpu.sync_copy(msgv, num_s.at[didx.at[0, p]], add=True)
            pltpu.sync_copy(sigv, den_s.at[didx.at[0, p]], add=True)
        if want_e:
            for p in (0, 1):
                bxv, dxv, exv, cev, msgv, sigv, gsem, cesem, osem = sets[p]
                pltpu.make_async_copy(
                    cev, eij_h.at[pl.ds(0, CH), pl.ds(0, H)], osem).wait()
        return carry

    if want_e:
        init = tuple(jnp.zeros((16,), F32) for _ in range(8))
    else:
        init = 0
    fin = lax.fori_loop(0, NCHUNK // 2, body, init)

    if want_e:
        for j in range(4):
            statsv[0, pl.ds(j * 16, 16)] = fin[j]
            statsv[1, pl.ds(j * 16, 16)] = fin[4 + j]
        pltpu.sync_copy(statsv, stats_h.at[c, s])

    plsc.subcore_barrier()
    for k in range(RPT // CH):
        r0 = s * RPT + k * CH
        pltpu.sync_copy(num_s.at[pl.ds(r0, CH)], msgv0)
        pltpu.sync_copy(msgv0, num_h.at[c, pl.ds(r0, CH)])
        pltpu.sync_copy(den_s.at[pl.ds(r0, CH)], msgv0)
        pltpu.sync_copy(msgv0, den_h.at[c, pl.ds(r0, CH)])


def _make_sc_kernel(want_e):
    outs = []
    if want_e:
        outs.append(jax.ShapeDtypeStruct((E, D), F32))
    outs.append(jax.ShapeDtypeStruct((2, NPAD, H), F32))
    outs.append(jax.ShapeDtypeStruct((2, NPAD, H), F32))
    if want_e:
        outs.append(jax.ShapeDtypeStruct((2, NT, 2, H), F32))
    bufset = [
        pltpu.VMEM((CH, H), F32),
        pltpu.VMEM((CH, H), F32),
        pltpu.VMEM((CH, H), F32),
        pltpu.VMEM((CH, H), F32),
        pltpu.VMEM((CH, H), F32),
        pltpu.VMEM((CH, H), F32),
    ]
    scratch = bufset + bufset + [
        pltpu.VMEM((2 * CH,), jnp.int32),
        pltpu.VMEM((1, 2, CH), jnp.int32),
        pltpu.VMEM((2, H), F32),
        pltpu.VMEM_SHARED((NPAD, H), F32),
        pltpu.VMEM_SHARED((NPAD, H), F32),
        pltpu.SemaphoreType.DMA,
        pltpu.SemaphoreType.DMA,
        pltpu.SemaphoreType.DMA,
        pltpu.SemaphoreType.DMA,
        pltpu.SemaphoreType.DMA,
        pltpu.SemaphoreType.DMA,
    ]
    mesh = plsc.VectorSubcoreMesh(core_axis_name="c", subcore_axis_name="s")
    return pl.kernel(
        functools.partial(_sc_body, want_e),
        out_type=tuple(outs),
        mesh=mesh,
        scratch_types=scratch,
        compiler_params=pltpu.CompilerParams(use_tc_tiling_on_sc=False),
    )


NBLK = 1000
EBLK = 4000


def _gelu(x):
    return 0.5 * x * (1.0 + lax.erf(x * 0.7071067811865476))


def _proj_body(x_ref, w_ref, b_ref, ax_ref, bx_ref, dx_ref, ex_ref):
    p = jnp.dot(x_ref[...], w_ref[...], preferred_element_type=F32) + b_ref[...]
    ax_ref[...] = p[:, :D]
    for k, ref in ((1, bx_ref), (2, dx_ref), (3, ex_ref)):
        half = p[:, k * D:(k + 1) * D]
        ref[...] = jnp.stack([half[:, :H], half[:, H:]], axis=0)


def _node_proj(x, wcat, bcat):
    grid = N // NBLK
    return pl.pallas_call(
        _proj_body,
        grid=(grid,),
        in_specs=[
            pl.BlockSpec((NBLK, D), lambda i: (i, 0)),
            pl.BlockSpec((D, 4 * D), lambda i: (0, 0)),
            pl.BlockSpec((1, 4 * D), lambda i: (0, 0)),
        ],
        out_specs=[
            pl.BlockSpec((NBLK, D), lambda i: (i, 0)),
            pl.BlockSpec((2, NBLK, H), lambda i: (0, i, 0)),
            pl.BlockSpec((2, NBLK, H), lambda i: (0, i, 0)),
            pl.BlockSpec((2, NBLK, H), lambda i: (0, i, 0)),
        ],
        out_shape=[
            jax.ShapeDtypeStruct((N, D), F32),
            jax.ShapeDtypeStruct((2, N, H), F32),
            jax.ShapeDtypeStruct((2, N, H), F32),
            jax.ShapeDtypeStruct((2, N, H), F32),
        ],
    )(x, wcat, bcat)


def _bond_ce_body(attr_ref, tbl_ref, wc_ref, bc_ref, e0_ref, ce_ref):
    attr = attr_ref[...]
    ohs = []
    for k in range(3):
        iota = lax.broadcasted_iota(jnp.int32, (1, 5), 1)
        ohs.append((attr[:, k:k + 1] == iota).astype(F32))
    oh = jnp.concatenate(ohs, axis=1)
    e0 = jnp.dot(oh, tbl_ref[...], preferred_element_type=F32)
    e0_ref[...] = e0
    ce_ref[...] = jnp.dot(e0, wc_ref[...], preferred_element_type=F32) + bc_ref[...]


def _bond_ce(edge_attr, tblcat, wc, bc):
    grid = E // EBLK
    return pl.pallas_call(
        _bond_ce_body,
        grid=(grid,),
        in_specs=[
            pl.BlockSpec((EBLK, 3), lambda i: (i, 0)),
            pl.BlockSpec((15, D), lambda i: (0, 0)),
            pl.BlockSpec((D, D), lambda i: (0, 0)),
            pl.BlockSpec((1, D), lambda i: (0, 0)),
        ],
        out_specs=[
            pl.BlockSpec((EBLK, D), lambda i: (i, 0)),
            pl.BlockSpec((EBLK, D), lambda i: (i, 0)),
        ],
        out_shape=[
            jax.ShapeDtypeStruct((E, D), F32),
            jax.ShapeDtypeStruct((E, D), F32),
        ],
    )(edge_attr, tblcat, wc, bc)


def _edge_update_ce_body(eij_ref, ep_ref, st_ref, g_ref, b_ref, wc_ref,
                         bc_ref, e_ref, ce_ref):
    st = st_ref[...]
    sums = jnp.sum(st[:, :, 0, :], axis=1)
    sqs = jnp.sum(st[:, :, 1, :], axis=1)
    mu = jnp.concatenate([sums[0], sums[1]])[None, :] * (1.0 / E)
    var = jnp.concatenate([sqs[0], sqs[1]])[None, :] * (1.0 / E) - mu * mu
    inv = lax.rsqrt(var + 1e-5)
    xn = (eij_ref[...] - mu) * inv * g_ref[...] + b_ref[...]
    xn = _gelu(xn)
    enew = ep_ref[...] + xn
    e_ref[...] = enew
    ce_ref[...] = jnp.dot(enew, wc_ref[...], preferred_element_type=F32) + bc_ref[...]


def _edge_update_ce(eij, e_prev, stats, gamma, beta, wc, bc):
    grid = E // EBLK
    return pl.pallas_call(
        _edge_update_ce_body,
        grid=(grid,),
        in_specs=[
            pl.BlockSpec((EBLK, D), lambda i: (i, 0)),
            pl.BlockSpec((EBLK, D), lambda i: (i, 0)),
            pl.BlockSpec((2, NT, 2, H), lambda i: (0, 0, 0, 0)),
            pl.BlockSpec((1, D), lambda i: (0, 0)),
            pl.BlockSpec((1, D), lambda i: (0, 0)),
            pl.BlockSpec((D, D), lambda i: (0, 0)),
            pl.BlockSpec((1, D), lambda i: (0, 0)),
        ],
        out_specs=[
            pl.BlockSpec((EBLK, D), lambda i: (i, 0)),
            pl.BlockSpec((EBLK, D), lambda i: (i, 0)),
        ],
        out_shape=[
            jax.ShapeDtypeStruct((E, D), F32),
            jax.ShapeDtypeStruct((E, D), F32),
        ],
    )(eij, e_prev, stats, gamma, beta, wc, bc)


def _node_update_body(ax_ref, num_ref, den_ref, xin_ref, g_ref, b_ref,
                      out_ref):
    num0 = num_ref[0, :N, :]
    num1 = num_ref[1, :N, :]
    den0 = den_ref[0, :N, :]
    den1 = den_ref[1, :N, :]
    aggr = jnp.concatenate(
        [num0 / (den0 + 1e-6), num1 / (den1 + 1e-6)], axis=1)
    xn = ax_ref[...] + aggr
    mu = jnp.mean(xn, axis=0, keepdims=True)
    var = jnp.mean((xn - mu) ** 2, axis=0, keepdims=True)
    xn = (xn - mu) * lax.rsqrt(var + 1e-5) * g_ref[...] + b_ref[...]
    out_ref[...] = xin_ref[...] + _gelu(xn)


def _node_update(ax, num, den, x_in, gamma, beta):
    return pl.pallas_call(
        _node_update_body,
        out_shape=jax.ShapeDtypeStruct((N, D), F32),
    )(ax, num, den, x_in, gamma, beta)


def kernel(X_n, edge_index, edge_attr, PE, params):
    src = edge_index[0]
    dst = edge_index[1]
    tblcat = params["bond_tables"].reshape(3 * 5, D)
    layers = params["layers"]

    es, ce = _bond_ce(edge_attr, tblcat, layers[0]["WC"],
                      layers[0]["bC"][None, :])
    x = X_n
    for l, lp in enumerate(layers):
        wcat = jnp.concatenate([lp["WA"], lp["WB"], lp["WD"], lp["WE"]],
                               axis=1)
        bcat = jnp.concatenate([lp["bA"], lp["bB"], lp["bD"], lp["bE"]])[None, :]
        ax, bx3, dx3, ex3 = _node_proj(x, wcat, bcat)
        want_e = l + 1 < len(layers)
        sc = _make_sc_kernel(want_e)
        dst2 = dst.reshape(NT * (NCHUNK // 2), 2, CH)
        if want_e:
            eijs, num, den, stats = sc(src, dst2, bx3, dx3, ex3, ce)
        else:
            num, den = sc(src, dst2, bx3, dx3, ex3, ce)
        x = _node_update(ax, num, den, x, lp["gamma_x"][None, :],
                         lp["beta_x"][None, :])
        if want_e:
            nlp = layers[l + 1]
            es, ce = _edge_update_ce(eijs, es, stats, lp["gamma_e"][None, :],
                                     lp["beta_e"][None, :], nlp["WC"],
                                     nlp["bC"][None, :])
    return x


# speedup vs baseline: 2.5761x; 1.0304x over previous
import functools
import jax
import jax.numpy as jnp
from jax import lax
from jax.experimental import pallas as pl
from jax.experimental.pallas import tpu as pltpu
from jax.experimental.pallas import tpu_sc as plsc

N = 10000
E = 320000
D = 128
H = 64
NT = 16
EPT = E // NT
CH = 40
NCHUNK = EPT // CH
NPAD = 10240
RPT = NPAD // NT
F32 = jnp.float32


def _half_cols(c, hbm_ref, off, n, vbuf, to_hbm=False, sem=None):
    for cc in (0, 1):
        @pl.when(c == cc)
        def _():
            sl = hbm_ref.at[pl.ds(off, n), pl.ds(cc * H, H)]
            if sem is None:
                if to_hbm:
                    pltpu.sync_copy(vbuf, sl)
                else:
                    pltpu.sync_copy(sl, vbuf)
            elif to_hbm:
                pltpu.async_copy(vbuf, sl, sem)
            else:
                pltpu.async_copy(sl, vbuf, sem)


def _sc_body(want_e, src_h, dst2_h, bx_h, dx_h, ex_h, ce_h, *rest):
    if want_e:
        eij_h, num_h, den_h, stats_h = rest[:4]
        scr = rest[4:]
    else:
        num_h, den_h = rest[:2]
        scr = rest[2:]
    (bxv0, dxv0, exv0, cev0, bxv1, dxv1, exv1, cev1, msg80, sig80,
     sidx, didx, statsv, num_s, den_s,
     gsem0, gsem1, cesem0, cesem1, osem0, osem1) = scr

    c = lax.axis_index("c")
    s = lax.axis_index("s")

    def zrow(r, carry):
        for j in range(4):
            msg80[r, pl.ds(j * 16, 16)] = jnp.zeros((16,), F32)
        return carry

    lax.fori_loop(0, 2 * CH, zrow, 0)
    for k in range(RPT // (2 * CH)):
        r0 = s * RPT + k * 2 * CH
        pltpu.sync_copy(msg80, num_s.at[pl.ds(r0, 2 * CH)])
        pltpu.sync_copy(msg80, den_s.at[pl.ds(r0, 2 * CH)])
    plsc.subcore_barrier()

    base = s * EPT
    bxc = bx_h.at[c]
    dxc = dx_h.at[c]
    exc = ex_h.at[c]
    sets = (
        (bxv0, dxv0, exv0, cev0, gsem0, cesem0, osem0),
        (bxv1, dxv1, exv1, cev1, gsem1, cesem1, osem1),
    )

    def make_row(st, poff):
        bxv, dxv, exv, cev = st[:4]

        def row(r, rc):
            out = rc
            if want_e:
                sums = list(rc[:4])
                sqs = list(rc[4:])
            for j in range(4):
                sl = pl.ds(j * 16, 16)
                eij = dxv[r, sl] + exv[r, sl] + cev[r, sl]
                sig = 1.0 / (1.0 + jnp.exp(-eij))
                msg80[poff + r, sl] = sig * bxv[r, sl]
                sig80[poff + r, sl] = sig
                cev[r, sl] = eij
                if want_e:
                    sums[j] = sums[j] + eij
                    sqs[j] = sqs[j] + eij * eij
            if want_e:
                out = tuple(sums) + tuple(sqs)
            return out

        return row

    rows = tuple(make_row(st, p * CH) for p, st in enumerate(sets))

    def body(k, carry):
        off0 = base + k * 2 * CH
        pltpu.sync_copy(src_h.at[pl.ds(off0, 2 * CH)], sidx)
        pltpu.sync_copy(dst2_h.at[pl.ds(s * (NCHUNK // 2) + k, 1)], didx)
        gd = []
        for p in (0, 1):
            bxv, dxv, exv, cev, gsem, cesem, osem = sets[p]
            idxp = sidx.at[pl.ds(p * CH, CH)]
            gd.append(pltpu.async_copy(bxc.at[idxp], bxv, gsem))
            gd.append(pltpu.async_copy(exc.at[idxp], exv, gsem))
            gd.append(pltpu.async_copy(
                dxc.at[didx.at[0, pl.ds(p * CH, CH)]], dxv, gsem))
            _half_cols(c, ce_h, off0 + p * CH, CH, cev, sem=cesem)
        for p in (0, 1):
            bxv, dxv, exv, cev, gsem, cesem, osem = sets[p]
            gd[3 * p].wait()
            gd[3 * p + 1].wait()
            gd[3 * p + 2].wait()
            pltpu.make_async_copy(
                ce_h.at[pl.ds(0, CH), pl.ds(0, H)], cev, cesem).wait()
            carry = lax.fori_loop(0, CH, rows[p], carry)
            if want_e:
                _half_cols(c, eij_h, off0 + p * CH, CH, cev, to_hbm=True,
                           sem=osem)
        pltpu.sync_copy(msg80, num_s.at[didx.at[0]], add=True)
        pltpu.sync_copy(sig80, den_s.at[didx.at[0]], add=True)
        if want_e:
            for p in (0, 1):
                bxv, dxv, exv, cev, gsem, cesem, osem = sets[p]
                pltpu.make_async_copy(
                    cev, eij_h.at[pl.ds(0, CH), pl.ds(0, H)], osem).wait()
        return carry

    if want_e:
        init = tuple(jnp.zeros((16,), F32) for _ in range(8))
    else:
        init = 0
    fin = lax.fori_loop(0, NCHUNK // 2, body, init)

    if want_e:
        for j in range(4):
            statsv[0, pl.ds(j * 16, 16)] = fin[j]
            statsv[1, pl.ds(j * 16, 16)] = fin[4 + j]
        pltpu.sync_copy(statsv, stats_h.at[c, s])

    plsc.subcore_barrier()
    for k in range(RPT // (2 * CH)):
        r0 = s * RPT + k * 2 * CH
        pltpu.sync_copy(num_s.at[pl.ds(r0, 2 * CH)], msg80)
        pltpu.sync_copy(msg80, num_h.at[c, pl.ds(r0, 2 * CH)])
        pltpu.sync_copy(den_s.at[pl.ds(r0, 2 * CH)], msg80)
        pltpu.sync_copy(msg80, den_h.at[c, pl.ds(r0, 2 * CH)])


def _make_sc_kernel(want_e):
    outs = []
    if want_e:
        outs.append(jax.ShapeDtypeStruct((E, D), F32))
    outs.append(jax.ShapeDtypeStruct((2, NPAD, H), F32))
    outs.append(jax.ShapeDtypeStruct((2, NPAD, H), F32))
    if want_e:
        outs.append(jax.ShapeDtypeStruct((2, NT, 2, H), F32))
    bufset = [
        pltpu.VMEM((CH, H), F32),
        pltpu.VMEM((CH, H), F32),
        pltpu.VMEM((CH, H), F32),
        pltpu.VMEM((CH, H), F32),
    ]
    scratch = bufset + bufset + [
        pltpu.VMEM((2 * CH, H), F32),
        pltpu.VMEM((2 * CH, H), F32),
        pltpu.VMEM((2 * CH,), jnp.int32),
        pltpu.VMEM((1, 2 * CH), jnp.int32),
        pltpu.VMEM((2, H), F32),
        pltpu.VMEM_SHARED((NPAD, H), F32),
        pltpu.VMEM_SHARED((NPAD, H), F32),
        pltpu.SemaphoreType.DMA,
        pltpu.SemaphoreType.DMA,
        pltpu.SemaphoreType.DMA,
        pltpu.SemaphoreType.DMA,
        pltpu.SemaphoreType.DMA,
        pltpu.SemaphoreType.DMA,
    ]
    mesh = plsc.VectorSubcoreMesh(core_axis_name="c", subcore_axis_name="s")
    return pl.kernel(
        functools.partial(_sc_body, want_e),
        out_type=tuple(outs),
        mesh=mesh,
        scratch_types=scratch,
        compiler_params=pltpu.CompilerParams(use_tc_tiling_on_sc=False),
    )


NBLK = 1000
EBLK = 4000


def _gelu(x):
    return 0.5 * x * (1.0 + lax.erf(x * 0.7071067811865476))


def _proj_body(x_ref, w_ref, b_ref, ax_ref, bx_ref, dx_ref, ex_ref):
    p = jnp.dot(x_ref[...], w_ref[...], preferred_element_type=F32) + b_ref[...]
    ax_ref[...] = p[:, :D]
    for k, ref in ((1, bx_ref), (2, dx_ref), (3, ex_ref)):
        half = p[:, k * D:(k + 1) * D]
        ref[...] = jnp.stack([half[:, :H], half[:, H:]], axis=0)


def _node_proj(x, wcat, bcat):
    grid = N // NBLK
    return pl.pallas_call(
        _proj_body,
        grid=(grid,),
        in_specs=[
            pl.BlockSpec((NBLK, D), lambda i: (i, 0)),
            pl.BlockSpec((D, 4 * D), lambda i: (0, 0)),
            pl.BlockSpec((1, 4 * D), lambda i: (0, 0)),
        ],
        out_specs=[
            pl.BlockSpec((NBLK, D), lambda i: (i, 0)),
            pl.BlockSpec((2, NBLK, H), lambda i: (0, i, 0)),
            pl.BlockSpec((2, NBLK, H), lambda i: (0, i, 0)),
            pl.BlockSpec((2, NBLK, H), lambda i: (0, i, 0)),
        ],
        out_shape=[
            jax.ShapeDtypeStruct((N, D), F32),
            jax.ShapeDtypeStruct((2, N, H), F32),
            jax.ShapeDtypeStruct((2, N, H), F32),
            jax.ShapeDtypeStruct((2, N, H), F32),
        ],
    )(x, wcat, bcat)


def _bond_ce_body(attr_ref, tbl_ref, wc_ref, bc_ref, e0_ref, ce_ref):
    attr = attr_ref[...]
    ohs = []
    for k in range(3):
        iota = lax.broadcasted_iota(jnp.int32, (1, 5), 1)
        ohs.append((attr[:, k:k + 1] == iota).astype(F32))
    oh = jnp.concatenate(ohs, axis=1)
    e0 = jnp.dot(oh, tbl_ref[...], preferred_element_type=F32)
    e0_ref[...] = e0
    ce_ref[...] = jnp.dot(e0, wc_ref[...], preferred_element_type=F32) + bc_ref[...]


def _bond_ce(edge_attr, tblcat, wc, bc):
    grid = E // EBLK
    return pl.pallas_call(
        _bond_ce_body,
        grid=(grid,),
        in_specs=[
            pl.BlockSpec((EBLK, 3), lambda i: (i, 0)),
            pl.BlockSpec((15, D), lambda i: (0, 0)),
            pl.BlockSpec((D, D), lambda i: (0, 0)),
            pl.BlockSpec((1, D), lambda i: (0, 0)),
        ],
        out_specs=[
            pl.BlockSpec((EBLK, D), lambda i: (i, 0)),
            pl.BlockSpec((EBLK, D), lambda i: (i, 0)),
        ],
        out_shape=[
            jax.ShapeDtypeStruct((E, D), F32),
            jax.ShapeDtypeStruct((E, D), F32),
        ],
    )(edge_attr, tblcat, wc, bc)


def _edge_update_ce_body(eij_ref, ep_ref, st_ref, g_ref, b_ref, wc_ref,
                         bc_ref, e_ref, ce_ref):
    st = st_ref[...]
    sums = jnp.sum(st[:, :, 0, :], axis=1)
    sqs = jnp.sum(st[:, :, 1, :], axis=1)
    mu = jnp.concatenate([sums[0], sums[1]])[None, :] * (1.0 / E)
    var = jnp.concatenate([sqs[0], sqs[1]])[None, :] * (1.0 / E) - mu * mu
    inv = lax.rsqrt(var + 1e-5)
    xn = (eij_ref[...] - mu) * inv * g_ref[...] + b_ref[...]
    xn = _gelu(xn)
    enew = ep_ref[...] + xn
    e_ref[...] = enew
    ce_ref[...] = jnp.dot(enew, wc_ref[...], preferred_element_type=F32) + bc_ref[...]


def _edge_update_ce(eij, e_prev, stats, gamma, beta, wc, bc):
    grid = E // EBLK
    return pl.pallas_call(
        _edge_update_ce_body,
        grid=(grid,),
        in_specs=[
            pl.BlockSpec((EBLK, D), lambda i: (i, 0)),
            pl.BlockSpec((EBLK, D), lambda i: (i, 0)),
            pl.BlockSpec((2, NT, 2, H), lambda i: (0, 0, 0, 0)),
            pl.BlockSpec((1, D), lambda i: (0, 0)),
            pl.BlockSpec((1, D), lambda i: (0, 0)),
            pl.BlockSpec((D, D), lambda i: (0, 0)),
            pl.BlockSpec((1, D), lambda i: (0, 0)),
        ],
        out_specs=[
            pl.BlockSpec((EBLK, D), lambda i: (i, 0)),
            pl.BlockSpec((EBLK, D), lambda i: (i, 0)),
        ],
        out_shape=[
            jax.ShapeDtypeStruct((E, D), F32),
            jax.ShapeDtypeStruct((E, D), F32),
        ],
    )(eij, e_prev, stats, gamma, beta, wc, bc)


def _node_update_body(ax_ref, num_ref, den_ref, xin_ref, g_ref, b_ref,
                      out_ref):
    num0 = num_ref[0, :N, :]
    num1 = num_ref[1, :N, :]
    den0 = den_ref[0, :N, :]
    den1 = den_ref[1, :N, :]
    aggr = jnp.concatenate(
        [num0 / (den0 + 1e-6), num1 / (den1 + 1e-6)], axis=1)
    xn = ax_ref[...] + aggr
    mu = jnp.mean(xn, axis=0, keepdims=True)
    var = jnp.mean((xn - mu) ** 2, axis=0, keepdims=True)
    xn = (xn - mu) * lax.rsqrt(var + 1e-5) * g_ref[...] + b_ref[...]
    out_ref[...] = xin_ref[...] + _gelu(xn)


def _node_update(ax, num, den, x_in, gamma, beta):
    return pl.pallas_call(
        _node_update_body,
        out_shape=jax.ShapeDtypeStruct((N, D), F32),
    )(ax, num, den, x_in, gamma, beta)


def kernel(X_n, edge_index, edge_attr, PE, params):
    src = edge_index[0]
    dst = edge_index[1]
    tblcat = params["bond_tables"].reshape(3 * 5, D)
    layers = params["layers"]

    es, ce = _bond_ce(edge_attr, tblcat, layers[0]["WC"],
                      layers[0]["bC"][None, :])
    x = X_n
    for l, lp in enumerate(layers):
        wcat = jnp.concatenate([lp["WA"], lp["WB"], lp["WD"], lp["WE"]],
                               axis=1)
        bcat = jnp.concatenate([lp["bA"], lp["bB"], lp["bD"], lp["bE"]])[None, :]
        ax, bx3, dx3, ex3 = _node_proj(x, wcat, bcat)
        want_e = l + 1 < len(layers)
        sc = _make_sc_kernel(want_e)
        dst2 = dst.reshape(NT * (NCHUNK // 2), 2 * CH)
        if want_e:
            eijs, num, den, stats = sc(src, dst2, bx3, dx3, ex3, ce)
        else:
            num, den = sc(src, dst2, bx3, dx3, ex3, ce)
        x = _node_update(ax, num, den, x, lp["gamma_x"][None, :],
                         lp["beta_x"][None, :])
        if want_e:
            nlp = layers[l + 1]
            es, ce = _edge_update_ce(eijs, es, stats, lp["gamma_e"][None, :],
                                     lp["beta_e"][None, :], nlp["WC"],
                                     nlp["bC"][None, :])
    return x


# speedup vs baseline: 2.6910x; 1.0446x over previous
import functools
import jax
import jax.numpy as jnp
from jax import lax
from jax.experimental import pallas as pl
from jax.experimental.pallas import tpu as pltpu
from jax.experimental.pallas import tpu_sc as plsc

N = 10000
E = 320000
D = 128
H = 64
NT = 16
EPT = E // NT
CH = 40
NCHUNK = EPT // CH
NPAD = 10240
RPT = NPAD // NT
F32 = jnp.float32


def _half_cols(c, hbm_ref, off, n, vbuf, to_hbm=False, sem=None):
    for cc in (0, 1):
        @pl.when(c == cc)
        def _():
            sl = hbm_ref.at[pl.ds(off, n), pl.ds(cc * H, H)]
            if sem is None:
                if to_hbm:
                    pltpu.sync_copy(vbuf, sl)
                else:
                    pltpu.sync_copy(sl, vbuf)
            elif to_hbm:
                pltpu.async_copy(vbuf, sl, sem)
            else:
                pltpu.async_copy(sl, vbuf, sem)


def _sc_body(want_e, ei_h, bx_h, dx_h, ex_h, ce_h, *rest):
    if want_e:
        eij_h, num_h, den_h, stats_h = rest[:4]
        scr = rest[4:]
    else:
        num_h, den_h = rest[:2]
        scr = rest[2:]
    (bxv0, dxv0, exv0, cev0, bxv1, dxv1, exv1, cev1, msg80, sig80,
     eij80, idx2, statsv, num_s, den_s,
     gsem0, gsem1, cesem0, cesem1) = scr

    c = lax.axis_index("c")
    s = lax.axis_index("s")

    def zrow(r, carry):
        for j in range(4):
            msg80[r, pl.ds(j * 16, 16)] = jnp.zeros((16,), F32)
        return carry

    lax.fori_loop(0, 2 * CH, zrow, 0)
    for k in range(RPT // (2 * CH)):
        r0 = s * RPT + k * 2 * CH
        pltpu.sync_copy(msg80, num_s.at[pl.ds(r0, 2 * CH)])
        pltpu.sync_copy(msg80, den_s.at[pl.ds(r0, 2 * CH)])
    plsc.subcore_barrier()

    base = s * EPT
    bxc = bx_h.at[c]
    dxc = dx_h.at[c]
    exc = ex_h.at[c]
    sets = (
        (bxv0, dxv0, exv0, cev0, gsem0, cesem0),
        (bxv1, dxv1, exv1, cev1, gsem1, cesem1),
    )

    def make_row(st, poff):
        bxv, dxv, exv, cev, _, _ = st

        def row(r, rc):
            out = rc
            if want_e:
                sums = list(rc[:4])
                sqs = list(rc[4:])
            for j in range(4):
                sl = pl.ds(j * 16, 16)
                eij = dxv[r, sl] + exv[r, sl] + cev[r, sl]
                sig = 1.0 / (1.0 + jnp.exp(-eij))
                msg80[poff + r, sl] = sig * bxv[r, sl]
                sig80[poff + r, sl] = sig
                if want_e:
                    eij80[poff + r, sl] = eij
                if want_e:
                    sums[j] = sums[j] + eij
                    sqs[j] = sqs[j] + eij * eij
            if want_e:
                out = tuple(sums) + tuple(sqs)
            return out

        return row

    rows = tuple(make_row(st, p * CH) for p, st in enumerate(sets))

    def body(k, carry):
        off0 = base + k * 2 * CH
        pltpu.sync_copy(ei_h.at[:, pl.ds(off0, 2 * CH)], idx2)
        gd = []
        for p in (0, 1):
            bxv, dxv, exv, cev, gsem, cesem = sets[p]
            idxp = idx2.at[0, pl.ds(p * CH, CH)]
            gd.append(pltpu.async_copy(bxc.at[idxp], bxv, gsem))
            gd.append(pltpu.async_copy(exc.at[idxp], exv, gsem))
            gd.append(pltpu.async_copy(
                dxc.at[idx2.at[1, pl.ds(p * CH, CH)]], dxv, gsem))
            _half_cols(c, ce_h, off0 + p * CH, CH, cev, sem=cesem)
        for p in (0, 1):
            bxv, dxv, exv, cev, gsem, cesem = sets[p]
            gd[3 * p].wait()
            gd[3 * p + 1].wait()
            gd[3 * p + 2].wait()
            pltpu.make_async_copy(
                ce_h.at[pl.ds(0, CH), pl.ds(0, H)], cev, cesem).wait()
            carry = lax.fori_loop(0, CH, rows[p], carry)
        pltpu.sync_copy(msg80, num_s.at[idx2.at[1]], add=True)
        pltpu.sync_copy(sig80, den_s.at[idx2.at[1]], add=True)
        if want_e:
            _half_cols(c, eij_h, off0, 2 * CH, eij80, to_hbm=True)
        return carry

    if want_e:
        init = tuple(jnp.zeros((16,), F32) for _ in range(8))
    else:
        init = 0
    fin = lax.fori_loop(0, NCHUNK // 2, body, init)

    if want_e:
        for j in range(4):
            statsv[0, pl.ds(j * 16, 16)] = fin[j]
            statsv[1, pl.ds(j * 16, 16)] = fin[4 + j]
        pltpu.sync_copy(statsv, stats_h.at[c, s])

    plsc.subcore_barrier()
    for k in range(RPT // (2 * CH)):
        r0 = s * RPT + k * 2 * CH
        pltpu.sync_copy(num_s.at[pl.ds(r0, 2 * CH)], msg80)
        pltpu.sync_copy(msg80, num_h.at[c, pl.ds(r0, 2 * CH)])
        pltpu.sync_copy(den_s.at[pl.ds(r0, 2 * CH)], msg80)
        pltpu.sync_copy(msg80, den_h.at[c, pl.ds(r0, 2 * CH)])


def _make_sc_kernel(want_e):
    outs = []
    if want_e:
        outs.append(jax.ShapeDtypeStruct((E, D), F32))
    outs.append(jax.ShapeDtypeStruct((2, NPAD, H), F32))
    outs.append(jax.ShapeDtypeStruct((2, NPAD, H), F32))
    if want_e:
        outs.append(jax.ShapeDtypeStruct((2, NT, 2, H), F32))
    bufset = [
        pltpu.VMEM((CH, H), F32),
        pltpu.VMEM((CH, H), F32),
        pltpu.VMEM((CH, H), F32),
        pltpu.VMEM((CH, H), F32),
    ]
    scratch = bufset + bufset + [
        pltpu.VMEM((2 * CH, H), F32),
        pltpu.VMEM((2 * CH, H), F32),
        pltpu.VMEM((2 * CH, H), F32),
        pltpu.VMEM((2, 2 * CH), jnp.int32),
        pltpu.VMEM((2, H), F32),
        pltpu.VMEM_SHARED((NPAD, H), F32),
        pltpu.VMEM_SHARED((NPAD, H), F32),
        pltpu.SemaphoreType.DMA,
        pltpu.SemaphoreType.DMA,
        pltpu.SemaphoreType.DMA,
        pltpu.SemaphoreType.DMA,
    ]
    mesh = plsc.VectorSubcoreMesh(core_axis_name="c", subcore_axis_name="s")
    return pl.kernel(
        functools.partial(_sc_body, want_e),
        out_type=tuple(outs),
        mesh=mesh,
        scratch_types=scratch,
        compiler_params=pltpu.CompilerParams(use_tc_tiling_on_sc=False),
    )


NBLK = 1000
EBLK = 4000


def _gelu(x):
    return 0.5 * x * (1.0 + lax.erf(x * 0.7071067811865476))


def _proj_body(x_ref, w_ref, b_ref, ax_ref, bx_ref, dx_ref, ex_ref):
    p = jnp.dot(x_ref[...], w_ref[...], preferred_element_type=F32) + b_ref[...]
    ax_ref[...] = p[:, :D]
    for k, ref in ((1, bx_ref), (2, dx_ref), (3, ex_ref)):
        half = p[:, k * D:(k + 1) * D]
        ref[...] = jnp.stack([half[:, :H], half[:, H:]], axis=0)


def _node_proj(x, wcat, bcat):
    grid = N // NBLK
    return pl.pallas_call(
        _proj_body,
        grid=(grid,),
        in_specs=[
            pl.BlockSpec((NBLK, D), lambda i: (i, 0)),
            pl.BlockSpec((D, 4 * D), lambda i: (0, 0)),
            pl.BlockSpec((1, 4 * D), lambda i: (0, 0)),
        ],
        out_specs=[
            pl.BlockSpec((NBLK, D), lambda i: (i, 0)),
            pl.BlockSpec((2, NBLK, H), lambda i: (0, i, 0)),
            pl.BlockSpec((2, NBLK, H), lambda i: (0, i, 0)),
            pl.BlockSpec((2, NBLK, H), lambda i: (0, i, 0)),
        ],
        out_shape=[
            jax.ShapeDtypeStruct((N, D), F32),
            jax.ShapeDtypeStruct((2, N, H), F32),
            jax.ShapeDtypeStruct((2, N, H), F32),
            jax.ShapeDtypeStruct((2, N, H), F32),
        ],
    )(x, wcat, bcat)


def _bond_ce_body(attr_ref, tbl_ref, wc_ref, bc_ref, e0_ref, ce_ref):
    attr = attr_ref[...]
    ohs = []
    for k in range(3):
        iota = lax.broadcasted_iota(jnp.int32, (1, 5), 1)
        ohs.append((attr[:, k:k + 1] == iota).astype(F32))
    oh = jnp.concatenate(ohs, axis=1)
    e0 = jnp.dot(oh, tbl_ref[...], preferred_element_type=F32)
    e0_ref[...] = e0
    ce_ref[...] = jnp.dot(e0, wc_ref[...], preferred_element_type=F32) + bc_ref[...]


def _bond_ce(edge_attr, tblcat, wc, bc):
    grid = E // EBLK
    return pl.pallas_call(
        _bond_ce_body,
        grid=(grid,),
        in_specs=[
            pl.BlockSpec((EBLK, 3), lambda i: (i, 0)),
            pl.BlockSpec((15, D), lambda i: (0, 0)),
            pl.BlockSpec((D, D), lambda i: (0, 0)),
            pl.BlockSpec((1, D), lambda i: (0, 0)),
        ],
        out_specs=[
            pl.BlockSpec((EBLK, D), lambda i: (i, 0)),
            pl.BlockSpec((EBLK, D), lambda i: (i, 0)),
        ],
        out_shape=[
            jax.ShapeDtypeStruct((E, D), F32),
            jax.ShapeDtypeStruct((E, D), F32),
        ],
    )(edge_attr, tblcat, wc, bc)


def _edge_update_ce_body(eij_ref, ep_ref, st_ref, g_ref, b_ref, wc_ref,
                         bc_ref, e_ref, ce_ref):
    st = st_ref[...]
    sums = jnp.sum(st[:, :, 0, :], axis=1)
    sqs = jnp.sum(st[:, :, 1, :], axis=1)
    mu = jnp.concatenate([sums[0], sums[1]])[None, :] * (1.0 / E)
    var = jnp.concatenate([sqs[0], sqs[1]])[None, :] * (1.0 / E) - mu * mu
    inv = lax.rsqrt(var + 1e-5)
    xn = (eij_ref[...] - mu) * inv * g_ref[...] + b_ref[...]
    xn = _gelu(xn)
    enew = ep_ref[...] + xn
    e_ref[...] = enew
    ce_ref[...] = jnp.dot(enew, wc_ref[...], preferred_element_type=F32) + bc_ref[...]


def _edge_update_ce(eij, e_prev, stats, gamma, beta, wc, bc):
    grid = E // EBLK
    return pl.pallas_call(
        _edge_update_ce_body,
        grid=(grid,),
        in_specs=[
            pl.BlockSpec((EBLK, D), lambda i: (i, 0)),
            pl.BlockSpec((EBLK, D), lambda i: (i, 0)),
            pl.BlockSpec((2, NT, 2, H), lambda i: (0, 0, 0, 0)),
            pl.BlockSpec((1, D), lambda i: (0, 0)),
            pl.BlockSpec((1, D), lambda i: (0, 0)),
            pl.BlockSpec((D, D), lambda i: (0, 0)),
            pl.BlockSpec((1, D), lambda i: (0, 0)),
        ],
        out_specs=[
            pl.BlockSpec((EBLK, D), lambda i: (i, 0)),
            pl.BlockSpec((EBLK, D), lambda i: (i, 0)),
        ],
        out_shape=[
            jax.ShapeDtypeStruct((E, D), F32),
            jax.ShapeDtypeStruct((E, D), F32),
        ],
    )(eij, e_prev, stats, gamma, beta, wc, bc)


def _node_update_body(ax_ref, num_ref, den_ref, xin_ref, g_ref, b_ref,
                      out_ref):
    num0 = num_ref[0, :N, :]
    num1 = num_ref[1, :N, :]
    den0 = den_ref[0, :N, :]
    den1 = den_ref[1, :N, :]
    aggr = jnp.concatenate(
        [num0 / (den0 + 1e-6), num1 / (den1 + 1e-6)], axis=1)
    xn = ax_ref[...] + aggr
    mu = jnp.mean(xn, axis=0, keepdims=True)
    var = jnp.mean((xn - mu) ** 2, axis=0, keepdims=True)
    xn = (xn - mu) * lax.rsqrt(var + 1e-5) * g_ref[...] + b_ref[...]
    out_ref[...] = xin_ref[...] + _gelu(xn)


def _node_update(ax, num, den, x_in, gamma, beta):
    return pl.pallas_call(
        _node_update_body,
        out_shape=jax.ShapeDtypeStruct((N, D), F32),
    )(ax, num, den, x_in, gamma, beta)


def kernel(X_n, edge_index, edge_attr, PE, params):
    tblcat = params["bond_tables"].reshape(3 * 5, D)
    layers = params["layers"]

    es, ce = _bond_ce(edge_attr, tblcat, layers[0]["WC"],
                      layers[0]["bC"][None, :])
    x = X_n
    for l, lp in enumerate(layers):
        wcat = jnp.concatenate([lp["WA"], lp["WB"], lp["WD"], lp["WE"]],
                               axis=1)
        bcat = jnp.concatenate([lp["bA"], lp["bB"], lp["bD"], lp["bE"]])[None, :]
        ax, bx3, dx3, ex3 = _node_proj(x, wcat, bcat)
        want_e = l + 1 < len(layers)
        sc = _make_sc_kernel(want_e)
        if want_e:
            eijs, num, den, stats = sc(edge_index, bx3, dx3, ex3, ce)
        else:
            num, den = sc(edge_index, bx3, dx3, ex3, ce)
        x = _node_update(ax, num, den, x, lp["gamma_x"][None, :],
                         lp["beta_x"][None, :])
        if want_e:
            nlp = layers[l + 1]
            es, ce = _edge_update_ce(eijs, es, stats, lp["gamma_e"][None, :],
                                     lp["beta_e"][None, :], nlp["WC"],
                                     nlp["bC"][None, :])
    return x


# speedup vs baseline: 2.7074x; 1.0061x over previous
import functools
import jax
import jax.numpy as jnp
from jax import lax
from jax.experimental import pallas as pl
from jax.experimental.pallas import tpu as pltpu
from jax.experimental.pallas import tpu_sc as plsc

N = 10000
E = 320000
D = 128
H = 64
NT = 16
EPT = E // NT
CH = 40
NCHUNK = EPT // CH
NPAD = 10240
RPT = NPAD // NT
F32 = jnp.float32


def _half_cols(c, hbm_ref, off, n, vbuf, to_hbm=False, sem=None):
    for cc in (0, 1):
        @pl.when(c == cc)
        def _():
            sl = hbm_ref.at[pl.ds(off, n), pl.ds(cc * H, H)]
            if sem is None:
                if to_hbm:
                    pltpu.sync_copy(vbuf, sl)
                else:
                    pltpu.sync_copy(sl, vbuf)
            elif to_hbm:
                pltpu.async_copy(vbuf, sl, sem)
            else:
                pltpu.async_copy(sl, vbuf, sem)


def _sc_body(want_e, ei_h, bx_h, dx_h, ex_h, ce_h, *rest):
    if want_e:
        eij_h, num_h, den_h, stats_h = rest[:4]
        scr = rest[4:]
    else:
        num_h, den_h = rest[:2]
        scr = rest[2:]
    (bxv0, dxv0, exv0, cev0, bxv1, dxv1, exv1, cev1, msg80, sig80,
     eij80, idx2, statsv, num_s, den_s,
     gsem0, gsem1, cesem0, cesem1) = scr

    c = lax.axis_index("c")
    s = lax.axis_index("s")

    def zrow(r, carry):
        for j in range(4):
            msg80[r, pl.ds(j * 16, 16)] = jnp.zeros((16,), F32)
        return carry

    lax.fori_loop(0, 2 * CH, zrow, 0)
    for k in range(RPT // (2 * CH)):
        r0 = s * RPT + k * 2 * CH
        pltpu.sync_copy(msg80, num_s.at[pl.ds(r0, 2 * CH)])
        pltpu.sync_copy(msg80, den_s.at[pl.ds(r0, 2 * CH)])
    plsc.subcore_barrier()

    base = s * EPT
    bxc = bx_h.at[c]
    dxc = dx_h.at[c]
    exc = ex_h.at[c]
    sets = (
        (bxv0, dxv0, exv0, cev0, gsem0, cesem0),
        (bxv1, dxv1, exv1, cev1, gsem1, cesem1),
    )

    def make_row(st, poff):
        bxv, dxv, exv, cev, _, _ = st

        def row(r, rc):
            out = rc
            if want_e:
                sums = list(rc[:4])
                sqs = list(rc[4:])
            for j in range(4):
                sl = pl.ds(j * 16, 16)
                eij = dxv[r, sl] + exv[r, sl] + cev[r, sl]
                sig = 1.0 / (1.0 + jnp.exp(-eij))
                msg80[poff + r, sl] = sig * bxv[r, sl]
                sig80[poff + r, sl] = sig
                if want_e:
                    eij80[poff + r, sl] = eij
                if want_e:
                    sums[j] = sums[j] + eij
                    sqs[j] = sqs[j] + eij * eij
            if want_e:
                out = tuple(sums) + tuple(sqs)
            return out

        return row

    rows = tuple(make_row(st, p * CH) for p, st in enumerate(sets))

    def body(k, carry):
        off0 = base + k * 2 * CH
        pltpu.sync_copy(ei_h.at[:, pl.ds(off0, 2 * CH)], idx2)
        gd = []
        for p in (0, 1):
            bxv, dxv, exv, cev, gsem, cesem = sets[p]
            idxp = idx2.at[0, pl.ds(p * CH, CH)]
            gd.append(pltpu.async_copy(bxc.at[idxp], bxv, gsem))
            gd.append(pltpu.async_copy(exc.at[idxp], exv, gsem))
            gd.append(pltpu.async_copy(
                dxc.at[idx2.at[1, pl.ds(p * CH, CH)]], dxv, gsem))
            _half_cols(c, ce_h, off0 + p * CH, CH, cev, sem=cesem)
        for p in (0, 1):
            bxv, dxv, exv, cev, gsem, cesem = sets[p]
            gd[3 * p].wait()
            gd[3 * p + 1].wait()
            gd[3 * p + 2].wait()
            pltpu.make_async_copy(
                ce_h.at[pl.ds(0, CH), pl.ds(0, H)], cev, cesem).wait()
            carry = lax.fori_loop(0, CH, rows[p], carry)
        pltpu.sync_copy(msg80, num_s.at[idx2.at[1]], add=True)
        pltpu.sync_copy(sig80, den_s.at[idx2.at[1]], add=True)
        if want_e:
            _half_cols(c, eij_h, off0, 2 * CH, eij80, to_hbm=True)
        return carry

    if want_e:
        init = tuple(jnp.zeros((16,), F32) for _ in range(8))
    else:
        init = 0
    fin = lax.fori_loop(0, NCHUNK // 2, body, init)

    if want_e:
        for j in range(4):
            statsv[0, pl.ds(j * 16, 16)] = fin[j]
            statsv[1, pl.ds(j * 16, 16)] = fin[4 + j]
        pltpu.sync_copy(statsv, stats_h.at[c, s])

    plsc.subcore_barrier()
    for k in range(RPT // (2 * CH)):
        r0 = s * RPT + k * 2 * CH
        pltpu.sync_copy(num_s.at[pl.ds(r0, 2 * CH)], msg80)
        pltpu.sync_copy(msg80, num_h.at[c, pl.ds(r0, 2 * CH)])
        pltpu.sync_copy(den_s.at[pl.ds(r0, 2 * CH)], msg80)
        pltpu.sync_copy(msg80, den_h.at[c, pl.ds(r0, 2 * CH)])


def _make_sc_kernel(want_e):
    outs = []
    if want_e:
        outs.append(jax.ShapeDtypeStruct((E, D), F32))
    outs.append(jax.ShapeDtypeStruct((2, NPAD, H), F32))
    outs.append(jax.ShapeDtypeStruct((2, NPAD, H), F32))
    if want_e:
        outs.append(jax.ShapeDtypeStruct((2, NT, 2, H), F32))
    bufset = [
        pltpu.VMEM((CH, H), F32),
        pltpu.VMEM((CH, H), F32),
        pltpu.VMEM((CH, H), F32),
        pltpu.VMEM((CH, H), F32),
    ]
    scratch = bufset + bufset + [
        pltpu.VMEM((2 * CH, H), F32),
        pltpu.VMEM((2 * CH, H), F32),
        pltpu.VMEM((2 * CH, H), F32),
        pltpu.VMEM((2, 2 * CH), jnp.int32),
        pltpu.VMEM((2, H), F32),
        pltpu.VMEM_SHARED((NPAD, H), F32),
        pltpu.VMEM_SHARED((NPAD, H), F32),
        pltpu.SemaphoreType.DMA,
        pltpu.SemaphoreType.DMA,
        pltpu.SemaphoreType.DMA,
        pltpu.SemaphoreType.DMA,
    ]
    mesh = plsc.VectorSubcoreMesh(core_axis_name="c", subcore_axis_name="s")
    return pl.kernel(
        functools.partial(_sc_body, want_e),
        out_type=tuple(outs),
        mesh=mesh,
        scratch_types=scratch,
        compiler_params=pltpu.CompilerParams(use_tc_tiling_on_sc=False),
    )


NBLK = 1000
EBLK = 8000


def _gelu(x):
    return 0.5 * x * (1.0 + lax.erf(x * 0.7071067811865476))


def _proj_body(x_ref, w_ref, b_ref, ax_ref, bx_ref, dx_ref, ex_ref):
    p = jnp.dot(x_ref[...], w_ref[...], preferred_element_type=F32) + b_ref[...]
    ax_ref[...] = p[:, :D]
    for k, ref in ((1, bx_ref), (2, dx_ref), (3, ex_ref)):
        half = p[:, k * D:(k + 1) * D]
        ref[...] = jnp.stack([half[:, :H], half[:, H:]], axis=0)


def _node_proj(x, wcat, bcat):
    grid = N // NBLK
    return pl.pallas_call(
        _proj_body,
        grid=(grid,),
        in_specs=[
            pl.BlockSpec((NBLK, D), lambda i: (i, 0)),
            pl.BlockSpec((D, 4 * D), lambda i: (0, 0)),
            pl.BlockSpec((1, 4 * D), lambda i: (0, 0)),
        ],
        out_specs=[
            pl.BlockSpec((NBLK, D), lambda i: (i, 0)),
            pl.BlockSpec((2, NBLK, H), lambda i: (0, i, 0)),
            pl.BlockSpec((2, NBLK, H), lambda i: (0, i, 0)),
            pl.BlockSpec((2, NBLK, H), lambda i: (0, i, 0)),
        ],
        out_shape=[
            jax.ShapeDtypeStruct((N, D), F32),
            jax.ShapeDtypeStruct((2, N, H), F32),
            jax.ShapeDtypeStruct((2, N, H), F32),
            jax.ShapeDtypeStruct((2, N, H), F32),
        ],
    )(x, wcat, bcat)


def _bond_ce_body(attr_ref, tbl_ref, wc_ref, bc_ref, e0_ref, ce_ref):
    attr = attr_ref[...]
    ohs = []
    for k in range(3):
        iota = lax.broadcasted_iota(jnp.int32, (1, 5), 1)
        ohs.append((attr[:, k:k + 1] == iota).astype(F32))
    oh = jnp.concatenate(ohs, axis=1)
    e0 = jnp.dot(oh, tbl_ref[...], preferred_element_type=F32)
    e0_ref[...] = e0
    ce_ref[...] = jnp.dot(e0, wc_ref[...], preferred_element_type=F32) + bc_ref[...]


def _bond_ce(edge_attr, tblcat, wc, bc):
    grid = E // EBLK
    return pl.pallas_call(
        _bond_ce_body,
        grid=(grid,),
        in_specs=[
            pl.BlockSpec((EBLK, 3), lambda i: (i, 0)),
            pl.BlockSpec((15, D), lambda i: (0, 0)),
            pl.BlockSpec((D, D), lambda i: (0, 0)),
            pl.BlockSpec((1, D), lambda i: (0, 0)),
        ],
        out_specs=[
            pl.BlockSpec((EBLK, D), lambda i: (i, 0)),
            pl.BlockSpec((EBLK, D), lambda i: (i, 0)),
        ],
        out_shape=[
            jax.ShapeDtypeStruct((E, D), F32),
            jax.ShapeDtypeStruct((E, D), F32),
        ],
    )(edge_attr, tblcat, wc, bc)


def _edge_update_ce_body(eij_ref, ep_ref, st_ref, g_ref, b_ref, wc_ref,
                         bc_ref, e_ref, ce_ref):
    st = st_ref[...]
    sums = jnp.sum(st[:, :, 0, :], axis=1)
    sqs = jnp.sum(st[:, :, 1, :], axis=1)
    mu = jnp.concatenate([sums[0], sums[1]])[None, :] * (1.0 / E)
    var = jnp.concatenate([sqs[0], sqs[1]])[None, :] * (1.0 / E) - mu * mu
    inv = lax.rsqrt(var + 1e-5)
    xn = (eij_ref[...] - mu) * inv * g_ref[...] + b_ref[...]
    xn = _gelu(xn)
    enew = ep_ref[...] + xn
    e_ref[...] = enew
    ce_ref[...] = jnp.dot(enew, wc_ref[...], preferred_element_type=F32) + bc_ref[...]


def _edge_update_ce(eij, e_prev, stats, gamma, beta, wc, bc):
    grid = E // EBLK
    return pl.pallas_call(
        _edge_update_ce_body,
        grid=(grid,),
        in_specs=[
            pl.BlockSpec((EBLK, D), lambda i: (i, 0)),
            pl.BlockSpec((EBLK, D), lambda i: (i, 0)),
            pl.BlockSpec((2, NT, 2, H), lambda i: (0, 0, 0, 0)),
            pl.BlockSpec((1, D), lambda i: (0, 0)),
            pl.BlockSpec((1, D), lambda i: (0, 0)),
            pl.BlockSpec((D, D), lambda i: (0, 0)),
            pl.BlockSpec((1, D), lambda i: (0, 0)),
        ],
        out_specs=[
            pl.BlockSpec((EBLK, D), lambda i: (i, 0)),
            pl.BlockSpec((EBLK, D), lambda i: (i, 0)),
        ],
        out_shape=[
            jax.ShapeDtypeStruct((E, D), F32),
            jax.ShapeDtypeStruct((E, D), F32),
        ],
    )(eij, e_prev, stats, gamma, beta, wc, bc)


def _node_update_body(ax_ref, num_ref, den_ref, xin_ref, g_ref, b_ref,
                      out_ref):
    num0 = num_ref[0, :N, :]
    num1 = num_ref[1, :N, :]
    den0 = den_ref[0, :N, :]
    den1 = den_ref[1, :N, :]
    aggr = jnp.concatenate(
        [num0 / (den0 + 1e-6), num1 / (den1 + 1e-6)], axis=1)
    xn = ax_ref[...] + aggr
    mu = jnp.mean(xn, axis=0, keepdims=True)
    var = jnp.mean((xn - mu) ** 2, axis=0, keepdims=True)
    xn = (xn - mu) * lax.rsqrt(var + 1e-5) * g_ref[...] + b_ref[...]
    out_ref[...] = xin_ref[...] + _gelu(xn)


def _node_update(ax, num, den, x_in, gamma, beta):
    return pl.pallas_call(
        _node_update_body,
        out_shape=jax.ShapeDtypeStruct((N, D), F32),
    )(ax, num, den, x_in, gamma, beta)


def kernel(X_n, edge_index, edge_attr, PE, params):
    tblcat = params["bond_tables"].reshape(3 * 5, D)
    layers = params["layers"]

    es, ce = _bond_ce(edge_attr, tblcat, layers[0]["WC"],
                      layers[0]["bC"][None, :])
    x = X_n
    for l, lp in enumerate(layers):
        wcat = jnp.concatenate([lp["WA"], lp["WB"], lp["WD"], lp["WE"]],
                               axis=1)
        bcat = jnp.concatenate([lp["bA"], lp["bB"], lp["bD"], lp["bE"]])[None, :]
        ax, bx3, dx3, ex3 = _node_proj(x, wcat, bcat)
        want_e = l + 1 < len(layers)
        sc = _make_sc_kernel(want_e)
        if want_e:
            eijs, num, den, stats = sc(edge_index, bx3, dx3, ex3, ce)
        else:
            num, den = sc(edge_index, bx3, dx3, ex3, ce)
        x = _node_update(ax, num, den, x, lp["gamma_x"][None, :],
                         lp["beta_x"][None, :])
        if want_e:
            nlp = layers[l + 1]
            es, ce = _edge_update_ce(eijs, es, stats, lp["gamma_e"][None, :],
                                     lp["beta_e"][None, :], nlp["WC"],
                                     nlp["bC"][None, :])
    return x


# speedup vs baseline: 3.6271x; 1.3397x over previous
import functools
import jax
import jax.numpy as jnp
from jax import lax
from jax.experimental import pallas as pl
from jax.experimental.pallas import tpu as pltpu
from jax.experimental.pallas import tpu_sc as plsc

N = 10000
E = 320000
D = 128
H = 64
NT = 16
EPT = E // NT
CH = 40
NCHUNK = EPT // CH
NPAD = 10240
RPT = NPAD // NT
F32 = jnp.float32


def _half_cols(c, hbm_ref, off, n, vbuf, to_hbm=False, sem=None):
    for cc in (0, 1):
        @pl.when(c == cc)
        def _():
            sl = hbm_ref.at[pl.ds(off, n), pl.ds(cc * H, H)]
            if sem is None:
                if to_hbm:
                    pltpu.sync_copy(vbuf, sl)
                else:
                    pltpu.sync_copy(sl, vbuf)
            elif to_hbm:
                pltpu.async_copy(vbuf, sl, sem)
            else:
                pltpu.async_copy(sl, vbuf, sem)


def _sc_body(want_e, ei_h, bx_h, dx_h, ex_h, ce_h, *rest):
    if want_e:
        eij_h, num_h, den_h, stats_h = rest[:4]
        scr = rest[4:]
    else:
        num_h, den_h = rest[:2]
        scr = rest[2:]
    (bxv0, dxv0, exv0, cev0, bxv1, dxv1, exv1, cev1, msg80, sig80,
     eij80, idx2a, idx2b, statsv, num_s, den_s,
     gsem0, gsem1, cesem0, cesem1, aisem) = scr

    c = lax.axis_index("c")
    s = lax.axis_index("s")

    def zrow(r, carry):
        for j in range(4):
            msg80[r, pl.ds(j * 16, 16)] = jnp.zeros((16,), F32)
        return carry

    lax.fori_loop(0, 2 * CH, zrow, 0)
    for k in range(RPT // (2 * CH)):
        r0 = s * RPT + k * 2 * CH
        pltpu.sync_copy(msg80, num_s.at[pl.ds(r0, 2 * CH)])
        pltpu.sync_copy(msg80, den_s.at[pl.ds(r0, 2 * CH)])
    plsc.subcore_barrier()

    base = s * EPT
    bxc = bx_h.at[c]
    dxc = dx_h.at[c]
    exc = ex_h.at[c]
    sets = (
        (bxv0, dxv0, exv0, cev0, gsem0, cesem0),
        (bxv1, dxv1, exv1, cev1, gsem1, cesem1),
    )

    def make_row(st, poff):
        bxv, dxv, exv, cev, _, _ = st

        def row(r, rc):
            out = rc
            if want_e:
                sums = list(rc[:4])
                sqs = list(rc[4:])
            for j in range(4):
                sl = pl.ds(j * 16, 16)
                eij = dxv[r, sl] + exv[r, sl] + cev[r, sl]
                sig = 1.0 / (1.0 + jnp.exp(-eij))
                msg80[poff + r, sl] = sig * bxv[r, sl]
                sig80[poff + r, sl] = sig
                if want_e:
                    eij80[poff + r, sl] = eij
                if want_e:
                    sums[j] = sums[j] + eij
                    sqs[j] = sqs[j] + eij * eij
            if want_e:
                out = tuple(sums) + tuple(sqs)
            return out

        return row

    rows = tuple(make_row(st, p * CH) for p, st in enumerate(sets))

    def fire(st, src_idx, dst_idx, off):
        bxv, dxv, exv, cev, gsem, cesem = st
        pltpu.async_copy(bxc.at[src_idx], bxv, gsem)
        pltpu.async_copy(exc.at[src_idx], exv, gsem)
        pltpu.async_copy(dxc.at[dst_idx], dxv, gsem)
        _half_cols(c, ce_h, off, CH, cev, sem=cesem)

    def wait_set(st):
        bxv, dxv, exv, cev, gsem, cesem = st
        pltpu.make_async_copy(bxc.at[idx2a.at[0, pl.ds(0, CH)]], bxv,
                              gsem).wait()
        pltpu.make_async_copy(exc.at[idx2a.at[0, pl.ds(0, CH)]], exv,
                              gsem).wait()
        pltpu.make_async_copy(dxc.at[idx2a.at[1, pl.ds(0, CH)]], dxv,
                              gsem).wait()
        pltpu.make_async_copy(
            ce_h.at[pl.ds(0, CH), pl.ds(0, H)], cev, cesem).wait()

    lastoff = base + EPT - 2 * CH

    def body(k, carry):
        off0 = base + k * 2 * CH
        for g in range(2 * CH // 16):
            for rr in (0, 1):
                sl = pl.ds(g * 16, 16)
                idx2a[rr, sl] = idx2b[rr, sl]
        wait_set(sets[0])
        offn = jnp.minimum(off0 + 2 * CH, lastoff)
        pltpu.async_copy(ei_h.at[:, pl.ds(offn, 2 * CH)], idx2b, aisem)
        fire(sets[1], idx2a.at[0, pl.ds(CH, CH)], idx2a.at[1, pl.ds(CH, CH)],
             off0 + CH)
        carry = lax.fori_loop(0, CH, rows[0], carry)
        pltpu.make_async_copy(ei_h.at[:, pl.ds(0, 2 * CH)], idx2b,
                              aisem).wait()
        fire(sets[0], idx2b.at[0, pl.ds(0, CH)], idx2b.at[1, pl.ds(0, CH)],
             jnp.minimum(off0 + 2 * CH, lastoff))
        wait_set(sets[1])
        carry = lax.fori_loop(0, CH, rows[1], carry)
        pltpu.sync_copy(msg80, num_s.at[idx2a.at[1]], add=True)
        pltpu.sync_copy(sig80, den_s.at[idx2a.at[1]], add=True)
        if want_e:
            _half_cols(c, eij_h, off0, 2 * CH, eij80, to_hbm=True)
        return carry

    if want_e:
        init = tuple(jnp.zeros((16,), F32) for _ in range(8))
    else:
        init = 0
    pltpu.sync_copy(ei_h.at[:, pl.ds(base, 2 * CH)], idx2b)
    fire(sets[0], idx2b.at[0, pl.ds(0, CH)], idx2b.at[1, pl.ds(0, CH)], base)
    fin = lax.fori_loop(0, NCHUNK // 2, body, init)
    wait_set(sets[0])

    if want_e:
        for j in range(4):
            statsv[0, pl.ds(j * 16, 16)] = fin[j]
            statsv[1, pl.ds(j * 16, 16)] = fin[4 + j]
        pltpu.sync_copy(statsv, stats_h.at[c, s])

    plsc.subcore_barrier()
    for k in range(RPT // (2 * CH)):
        r0 = s * RPT + k * 2 * CH
        pltpu.sync_copy(num_s.at[pl.ds(r0, 2 * CH)], msg80)
        pltpu.sync_copy(msg80, num_h.at[c, pl.ds(r0, 2 * CH)])
        pltpu.sync_copy(den_s.at[pl.ds(r0, 2 * CH)], msg80)
        pltpu.sync_copy(msg80, den_h.at[c, pl.ds(r0, 2 * CH)])


def _make_sc_kernel(want_e):
    outs = []
    if want_e:
        outs.append(jax.ShapeDtypeStruct((E, D), F32))
    outs.append(jax.ShapeDtypeStruct((2, NPAD, H), F32))
    outs.append(jax.ShapeDtypeStruct((2, NPAD, H), F32))
    if want_e:
        outs.append(jax.ShapeDtypeStruct((2, NT, 2, H), F32))
    bufset = [
        pltpu.VMEM((CH, H), F32),
        pltpu.VMEM((CH, H), F32),
        pltpu.VMEM((CH, H), F32),
        pltpu.VMEM((CH, H), F32),
    ]
    scratch = bufset + bufset + [
        pltpu.VMEM((2 * CH, H), F32),
        pltpu.VMEM((2 * CH, H), F32),
        pltpu.VMEM((2 * CH, H), F32),
        pltpu.VMEM((2, 2 * CH), jnp.int32),
        pltpu.VMEM((2, 2 * CH), jnp.int32),
        pltpu.VMEM((2, H), F32),
        pltpu.VMEM_SHARED((NPAD, H), F32),
        pltpu.VMEM_SHARED((NPAD, H), F32),
        pltpu.SemaphoreType.DMA,
        pltpu.SemaphoreType.DMA,
        pltpu.SemaphoreType.DMA,
        pltpu.SemaphoreType.DMA,
        pltpu.SemaphoreType.DMA,
    ]
    mesh = plsc.VectorSubcoreMesh(core_axis_name="c", subcore_axis_name="s")
    return pl.kernel(
        functools.partial(_sc_body, want_e),
        out_type=tuple(outs),
        mesh=mesh,
        scratch_types=scratch,
        compiler_params=pltpu.CompilerParams(use_tc_tiling_on_sc=False),
    )


NBLK = 1000
EBLK = 8000


def _gelu(x):
    return 0.5 * x * (1.0 + lax.erf(x * 0.7071067811865476))


def _proj_body(x_ref, w_ref, b_ref, ax_ref, bx_ref, dx_ref, ex_ref):
    p = jnp.dot(x_ref[...], w_ref[...], preferred_element_type=F32) + b_ref[...]
    ax_ref[...] = p[:, :D]
    for k, ref in ((1, bx_ref), (2, dx_ref), (3, ex_ref)):
        half = p[:, k * D:(k + 1) * D]
        ref[...] = jnp.stack([half[:, :H], half[:, H:]], axis=0)


def _node_proj(x, wcat, bcat):
    grid = N // NBLK
    return pl.pallas_call(
        _proj_body,
        grid=(grid,),
        in_specs=[
            pl.BlockSpec((NBLK, D), lambda i: (i, 0)),
            pl.BlockSpec((D, 4 * D), lambda i: (0, 0)),
            pl.BlockSpec((1, 4 * D), lambda i: (0, 0)),
        ],
        out_specs=[
            pl.BlockSpec((NBLK, D), lambda i: (i, 0)),
            pl.BlockSpec((2, NBLK, H), lambda i: (0, i, 0)),
            pl.BlockSpec((2, NBLK, H), lambda i: (0, i, 0)),
            pl.BlockSpec((2, NBLK, H), lambda i: (0, i, 0)),
        ],
        out_shape=[
            jax.ShapeDtypeStruct((N, D), F32),
            jax.ShapeDtypeStruct((2, N, H), F32),
            jax.ShapeDtypeStruct((2, N, H), F32),
            jax.ShapeDtypeStruct((2, N, H), F32),
        ],
    )(x, wcat, bcat)


def _bond_ce_body(attr_ref, tbl_ref, wc_ref, bc_ref, e0_ref, ce_ref):
    attr = attr_ref[...]
    ohs = []
    for k in range(3):
        iota = lax.broadcasted_iota(jnp.int32, (1, 5), 1)
        ohs.append((attr[:, k:k + 1] == iota).astype(F32))
    oh = jnp.concatenate(ohs, axis=1)
    e0 = jnp.dot(oh, tbl_ref[...], preferred_element_type=F32)
    e0_ref[...] = e0
    ce_ref[...] = jnp.dot(e0, wc_ref[...], preferred_element_type=F32) + bc_ref[...]


def _bond_ce(edge_attr, tblcat, wc, bc):
    grid = E // EBLK
    return pl.pallas_call(
        _bond_ce_body,
        grid=(grid,),
        in_specs=[
            pl.BlockSpec((EBLK, 3), lambda i: (i, 0)),
            pl.BlockSpec((15, D), lambda i: (0, 0)),
            pl.BlockSpec((D, D), lambda i: (0, 0)),
            pl.BlockSpec((1, D), lambda i: (0, 0)),
        ],
        out_specs=[
            pl.BlockSpec((EBLK, D), lambda i: (i, 0)),
            pl.BlockSpec((EBLK, D), lambda i: (i, 0)),
        ],
        out_shape=[
            jax.ShapeDtypeStruct((E, D), F32),
            jax.ShapeDtypeStruct((E, D), F32),
        ],
    )(edge_attr, tblcat, wc, bc)


def _edge_update_ce_body(eij_ref, ep_ref, st_ref, g_ref, b_ref, wc_ref,
                         bc_ref, e_ref, ce_ref):
    st = st_ref[...]
    sums = jnp.sum(st[:, :, 0, :], axis=1)
    sqs = jnp.sum(st[:, :, 1, :], axis=1)
    mu = jnp.concatenate([sums[0], sums[1]])[None, :] * (1.0 / E)
    var = jnp.concatenate([sqs[0], sqs[1]])[None, :] * (1.0 / E) - mu * mu
    inv = lax.rsqrt(var + 1e-5)
    xn = (eij_ref[...] - mu) * inv * g_ref[...] + b_ref[...]
    xn = _gelu(xn)
    enew = ep_ref[...] + xn
    e_ref[...] = enew
    ce_ref[...] = jnp.dot(enew, wc_ref[...], preferred_element_type=F32) + bc_ref[...]


def _edge_update_ce(eij, e_prev, stats, gamma, beta, wc, bc):
    grid = E // EBLK
    return pl.pallas_call(
        _edge_update_ce_body,
        grid=(grid,),
        in_specs=[
            pl.BlockSpec((EBLK, D), lambda i: (i, 0)),
            pl.BlockSpec((EBLK, D), lambda i: (i, 0)),
            pl.BlockSpec((2, NT, 2, H), lambda i: (0, 0, 0, 0)),
            pl.BlockSpec((1, D), lambda i: (0, 0)),
            pl.BlockSpec((1, D), lambda i: (0, 0)),
            pl.BlockSpec((D, D), lambda i: (0, 0)),
            pl.BlockSpec((1, D), lambda i: (0, 0)),
        ],
        out_specs=[
            pl.BlockSpec((EBLK, D), lambda i: (i, 0)),
            pl.BlockSpec((EBLK, D), lambda i: (i, 0)),
        ],
        out_shape=[
            jax.ShapeDtypeStruct((E, D), F32),
            jax.ShapeDtypeStruct((E, D), F32),
        ],
    )(eij, e_prev, stats, gamma, beta, wc, bc)


def _node_update_body(ax_ref, num_ref, den_ref, xin_ref, g_ref, b_ref,
                      out_ref):
    num0 = num_ref[0, :N, :]
    num1 = num_ref[1, :N, :]
    den0 = den_ref[0, :N, :]
    den1 = den_ref[1, :N, :]
    aggr = jnp.concatenate(
        [num0 / (den0 + 1e-6), num1 / (den1 + 1e-6)], axis=1)
    xn = ax_ref[...] + aggr
    mu = jnp.mean(xn, axis=0, keepdims=True)
    var = jnp.mean((xn - mu) ** 2, axis=0, keepdims=True)
    xn = (xn - mu) * lax.rsqrt(var + 1e-5) * g_ref[...] + b_ref[...]
    out_ref[...] = xin_ref[...] + _gelu(xn)


def _node_update(ax, num, den, x_in, gamma, beta):
    return pl.pallas_call(
        _node_update_body,
        out_shape=jax.ShapeDtypeStruct((N, D), F32),
    )(ax, num, den, x_in, gamma, beta)


def kernel(X_n, edge_index, edge_attr, PE, params):
    tblcat = params["bond_tables"].reshape(3 * 5, D)
    layers = params["layers"]

    es, ce = _bond_ce(edge_attr, tblcat, layers[0]["WC"],
                      layers[0]["bC"][None, :])
    x = X_n
    for l, lp in enumerate(layers):
        wcat = jnp.concatenate([lp["WA"], lp["WB"], lp["WD"], lp["WE"]],
                               axis=1)
        bcat = jnp.concatenate([lp["bA"], lp["bB"], lp["bD"], lp["bE"]])[None, :]
        ax, bx3, dx3, ex3 = _node_proj(x, wcat, bcat)
        want_e = l + 1 < len(layers)
        sc = _make_sc_kernel(want_e)
        if want_e:
            eijs, num, den, stats = sc(edge_index, bx3, dx3, ex3, ce)
        else:
            num, den = sc(edge_index, bx3, dx3, ex3, ce)
        x = _node_update(ax, num, den, x, lp["gamma_x"][None, :],
                         lp["beta_x"][None, :])
        if want_e:
            nlp = layers[l + 1]
            es, ce = _edge_update_ce(eijs, es, stats, lp["gamma_e"][None, :],
                                     lp["beta_e"][None, :], nlp["WC"],
                                     nlp["bC"][None, :])
    return x
